# Initial kernel scaffold; baseline (speedup 1.0000x reference)
import jax
import jax.numpy as jnp
from jax.experimental import pallas as pl


def kernel(input_ids, params):
    raise NotImplementedError("write your pallas kernel here")


# speedup vs baseline: 1.7963x; 1.7963x over previous
import functools
import numpy as np
import jax
import jax.numpy as jnp
from jax import lax
from jax.experimental import pallas as pl
from jax.experimental.pallas import tpu as pltpu, tpu_sc as plsc

B = 2
S = 2048
D = 768
F = 3072
L = 2
H = 12
E = 8
CAP = 512
HD = D // H
HF = HD // 2
T = B * S
MBLK = 512
NMB = T // MBLK
_HPG = 4

_SC_NW = 32
_SC_BPW = T // _SC_NW


def _sc_gather(table, idx):
  V, Dd = table.shape
  mesh = plsc.VectorSubcoreMesh(core_axis_name="c", subcore_axis_name="s")

  @functools.partial(
      pl.kernel, mesh=mesh,
      out_type=jax.ShapeDtypeStruct((T, Dd), jnp.float32),
      scratch_types=[
          pltpu.VMEM((_SC_BPW,), jnp.int32),
          pltpu.VMEM((_SC_BPW, Dd), jnp.float32),
          pltpu.SemaphoreType.DMA,
      ],
  )
  def k(table_hbm, idx_hbm, out_hbm, idx_v, rows_v, sem):
    wid = lax.axis_index("s") * 2 + lax.axis_index("c")
    base = wid * _SC_BPW
    pltpu.sync_copy(idx_hbm.at[pl.ds(base, _SC_BPW)], idx_v)
    pltpu.async_copy(table_hbm.at[idx_v], rows_v, sem).wait()
    pltpu.sync_copy(rows_v, out_hbm.at[pl.ds(base, _SC_BPW)])

  return k(table, idx)


def _sc_scatter(rows, dest, nrows):
  Dd = rows.shape[1]
  mesh = plsc.VectorSubcoreMesh(core_axis_name="c", subcore_axis_name="s")

  @functools.partial(
      pl.kernel, mesh=mesh,
      out_type=jax.ShapeDtypeStruct((nrows, Dd), jnp.float32),
      scratch_types=[
          pltpu.VMEM((_SC_BPW,), jnp.int32),
          pltpu.VMEM((_SC_BPW, Dd), jnp.float32),
          pltpu.SemaphoreType.DMA,
      ],
  )
  def k(rows_hbm, dest_hbm, out_hbm, idx_v, rows_v, sem):
    wid = lax.axis_index("s") * 2 + lax.axis_index("c")
    base = wid * _SC_BPW
    pltpu.sync_copy(dest_hbm.at[pl.ds(base, _SC_BPW)], idx_v)
    pltpu.sync_copy(rows_hbm.at[pl.ds(base, _SC_BPW)], rows_v)
    pltpu.async_copy(rows_v, out_hbm.at[idx_v], sem).wait()

  return k(rows, dest)


def _qkv_body(x_ref, wq_ref, wk_ref, wv_ref, bq_ref, bk_ref, bv_ref,
              sin_ref, cos_ref, q1_ref, q2_ref, k1_ref, k2_ref, v_ref):
  x = x_ref[...]
  s = sin_ref[...]
  c = cos_ref[...]

  q = jnp.dot(x, wq_ref[...], preferred_element_type=jnp.float32) + bq_ref[...]
  q1 = q[:, :D // 2]
  q2 = q[:, D // 2:]
  q1_ref[...] = q1 * c - q2 * s
  q2_ref[...] = q1 * s + q2 * c

  k = jnp.dot(x, wk_ref[...], preferred_element_type=jnp.float32) + bk_ref[...]
  k1 = k[:, :D // 2]
  k2 = k[:, D // 2:]
  k1_ref[...] = k1 * c - k2 * s
  k2_ref[...] = k1 * s + k2 * c

  v_ref[...] = jnp.dot(x, wv_ref[...], preferred_element_type=jnp.float32) + bv_ref[...]


def _qkv(x, wq_p, wk_p, wv, bq_p, bk_p, bv, sin_f, cos_f):
  half = jax.ShapeDtypeStruct((T, D // 2), jnp.float32)
  full = jax.ShapeDtypeStruct((T, D), jnp.float32)
  mspec = lambda n: pl.BlockSpec((MBLK, n), lambda i: (i, 0))
  wspec = pl.BlockSpec((D, D), lambda i: (0, 0))
  bspec = pl.BlockSpec((1, D), lambda i: (0, 0))
  return pl.pallas_call(
      _qkv_body,
      grid=(NMB,),
      in_specs=[mspec(D), wspec, wspec, wspec, bspec, bspec, bspec,
                mspec(D // 2), mspec(D // 2)],
      out_specs=[mspec(D // 2), mspec(D // 2), mspec(D // 2), mspec(D // 2),
                 mspec(D)],
      out_shape=[half, half, half, half, full],
      compiler_params=pltpu.CompilerParams(
          dimension_semantics=("arbitrary",)),
  )(x, wq_p, wk_p, wv, bq_p, bk_p, bv, sin_f, cos_f)


def _attn_body(q1_ref, q2_ref, k1_ref, k2_ref, v_ref, o_ref):
  q1 = q1_ref[0]
  q2 = q2_ref[0]
  k1 = k1_ref[0]
  k2 = k2_ref[0]
  dn = (((1,), (1,)), ((), ()))
  scale = np.float32(1.0 / np.sqrt(HD))
  for h in range(_HPG):
    hf = slice(h * HF, (h + 1) * HF)
    s = lax.dot_general(q1[:, hf], k1[:, hf], dn,
                        preferred_element_type=jnp.float32)
    s = s + lax.dot_general(q2[:, hf], k2[:, hf], dn,
                            preferred_element_type=jnp.float32)
    s = s * scale
    m = jnp.max(s, axis=-1, keepdims=True)
    p = jnp.exp(s - m)
    l = jnp.sum(p, axis=-1, keepdims=True)
    o = jnp.dot(p, v_ref[0, :, h * HD:(h + 1) * HD],
                preferred_element_type=jnp.float32)
    o_ref[0, :, h * HD:(h + 1) * HD] = o / l


def _attention(q1, q2, k1, k2, v):
  QBLK = 512
  nq = S // QBLK
  ng = H // _HPG
  q1 = q1.reshape(B, S, D // 2)
  q2 = q2.reshape(B, S, D // 2)
  k1 = k1.reshape(B, S, D // 2)
  k2 = k2.reshape(B, S, D // 2)
  v = v.reshape(B, S, D)
  GF = _HPG * HF
  GD = _HPG * HD
  qspec = pl.BlockSpec((1, QBLK, GF), lambda b, g, i: (b, i, g))
  kspec = pl.BlockSpec((1, S, GF), lambda b, g, i: (b, 0, g))
  vspec = pl.BlockSpec((1, S, GD), lambda b, g, i: (b, 0, g))
  ospec = pl.BlockSpec((1, QBLK, GD), lambda b, g, i: (b, i, g))
  out = pl.pallas_call(
      _attn_body,
      grid=(B, ng, nq),
      in_specs=[qspec, qspec, kspec, kspec, vspec],
      out_specs=ospec,
      out_shape=jax.ShapeDtypeStruct((B, S, D), jnp.float32),
      compiler_params=pltpu.CompilerParams(
          dimension_semantics=("arbitrary", "arbitrary", "arbitrary")),
  )(q1, q2, k1, k2, v)
  return out.reshape(T, D)


def _ln(t, g, b):
  m = jnp.mean(t, axis=-1, keepdims=True)
  d = t - m
  var = jnp.mean(d * d, axis=-1, keepdims=True)
  return d * lax.rsqrt(var + 1e-5) * g + b


def _proj_ln_body(a_ref, wo_ref, bo_ref, res_ref, g_ref, b_ref, o_ref):
  t = jnp.dot(a_ref[...], wo_ref[...], preferred_element_type=jnp.float32)
  t = t + bo_ref[...] + res_ref[...]
  o_ref[...] = _ln(t, g_ref[...], b_ref[...])


def _proj_ln(a, wo, bo, res, g, b):
  mspec = pl.BlockSpec((MBLK, D), lambda i: (i, 0))
  wspec = pl.BlockSpec((D, D), lambda i: (0, 0))
  bspec = pl.BlockSpec((1, D), lambda i: (0, 0))
  return pl.pallas_call(
      _proj_ln_body,
      grid=(NMB,),
      in_specs=[mspec, wspec, bspec, mspec, bspec, bspec],
      out_specs=mspec,
      out_shape=jax.ShapeDtypeStruct((T, D), jnp.float32),
      compiler_params=pltpu.CompilerParams(
          dimension_semantics=("arbitrary",)),
  )(a, wo, bo, res, g, b)


def _router_body(x_ref, wr_ref, logits_ref, eidx_ref, gatek_ref,
                 destd_ref, destc_ref, oh_ref, pos_ref):
  logits = jnp.dot(x_ref[...], wr_ref[...],
                   preferred_element_type=jnp.float32)
  logits_ref[...] = logits
  m = jnp.max(logits, axis=-1, keepdims=True)
  p = jnp.exp(logits - m)
  p = p / jnp.sum(p, axis=-1, keepdims=True)
  gate = jnp.max(p, axis=-1, keepdims=True)
  iota_e = lax.broadcasted_iota(jnp.int32, (T, E), 1)
  eidx = jnp.min(jnp.where(p == gate, iota_e, E), axis=-1,
                 keepdims=True)
  eidx_ref[...] = eidx
  oh_ref[...] = (iota_e == eidx).astype(jnp.float32)

  r = lax.broadcasted_iota(jnp.int32, (MBLK, MBLK), 0)
  cc = lax.broadcasted_iota(jnp.int32, (MBLK, MBLK), 1)
  tril = (r >= cc).astype(jnp.float32)

  def step(j, carry):
    blk = oh_ref[pl.ds(j * MBLK, MBLK), :]
    cum = jnp.dot(tril, blk, preferred_element_type=jnp.float32) + carry
    pos_ref[pl.ds(j * MBLK, MBLK), :] = (
        jnp.sum(cum * blk, axis=-1, keepdims=True) - 1.0)
    return carry + jnp.sum(blk, axis=0, keepdims=True)

  lax.fori_loop(0, NMB, step, jnp.zeros((1, E), jnp.float32))

  pos = pos_ref[...].astype(jnp.int32)
  keep = pos < CAP
  gatek_ref[...] = gate * keep.astype(jnp.float32)
  slot = eidx * CAP + pos
  tok = lax.broadcasted_iota(jnp.int32, (T, 1), 0)
  trash = T + tok // _SC_BPW
  destd_ref[...] = jnp.where(keep, slot, trash)
  destc_ref[...] = eidx * CAP + jnp.minimum(pos, CAP - 1)


def _router(x, wr):
  full = lambda shp, dt: jax.ShapeDtypeStruct(shp, dt)
  spec = lambda shp: pl.BlockSpec(shp, lambda: (0,) * len(shp))
  return pl.pallas_call(
      _router_body,
      in_specs=[spec((T, D)), spec((D, E))],
      out_specs=[spec((T, E)), spec((T, 1)), spec((T, 1)), spec((T, 1)),
                 spec((T, 1))],
      out_shape=[full((T, E), jnp.float32), full((T, 1), jnp.int32),
                 full((T, 1), jnp.float32), full((T, 1), jnp.int32),
                 full((T, 1), jnp.int32)],
      scratch_shapes=[pltpu.VMEM((T, E), jnp.float32),
                      pltpu.VMEM((T, 1), jnp.float32)],
  )(x, wr)


def _erf(x):
  a1, a2, a3, a4, a5 = (0.254829592, -0.284496736, 1.421413741,
                        -1.453152027, 1.061405429)
  sgn = jnp.sign(x)
  ax = jnp.abs(x)
  t = 1.0 / (1.0 + 0.3275911 * ax)
  poly = ((((a5 * t + a4) * t + a3) * t + a2) * t + a1) * t
  return sgn * (1.0 - poly * jnp.exp(-ax * ax))


def _gelu(x):
  return x * 0.5 * (1.0 + _erf(x * np.float32(1.0 / np.sqrt(2.0))))


def _ffn_body(xe_ref, w1_ref, b1_ref, w2_ref, b2_ref, ye_ref):
  f = pl.program_id(1)
  h = _gelu(jnp.dot(xe_ref[0], w1_ref[0],
                    preferred_element_type=jnp.float32) + b1_ref[0])
  acc = jnp.dot(h, w2_ref[0], preferred_element_type=jnp.float32)

  @pl.when(f == 0)
  def _():
    ye_ref[0] = acc + b2_ref[0]

  @pl.when(f != 0)
  def _():
    ye_ref[0] = ye_ref[0] + acc


def _ffn(xe, w1, b1, w2, b2):
  FBLK = 768
  nf = F // FBLK
  return pl.pallas_call(
      _ffn_body,
      grid=(E, nf),
      in_specs=[
          pl.BlockSpec((1, CAP, D), lambda e, f: (e, 0, 0)),
          pl.BlockSpec((1, D, FBLK), lambda e, f: (e, 0, f)),
          pl.BlockSpec((1, 1, FBLK), lambda e, f: (e, 0, f)),
          pl.BlockSpec((1, FBLK, D), lambda e, f: (e, f, 0)),
          pl.BlockSpec((1, 1, D), lambda e, f: (e, 0, 0)),
      ],
      out_specs=pl.BlockSpec((1, CAP, D), lambda e, f: (e, 0, 0)),
      out_shape=jax.ShapeDtypeStruct((E, CAP, D), jnp.float32),
      compiler_params=pltpu.CompilerParams(
          dimension_semantics=("arbitrary", "arbitrary")),
  )(xe, w1, b1, w2, b2)


def _combine_ln_body(y_ref, gk_ref, res_ref, g_ref, b_ref, o_ref):
  t = res_ref[...] + y_ref[...] * gk_ref[...]
  o_ref[...] = _ln(t, g_ref[...], b_ref[...])


def _combine_ln_final_body(y_ref, gk_ref, res_ref, g_ref, b_ref,
                           gf_ref, bf_ref, o_ref):
  t = res_ref[...] + y_ref[...] * gk_ref[...]
  t = _ln(t, g_ref[...], b_ref[...])
  o_ref[...] = _ln(t, gf_ref[...], bf_ref[...])


def _combine_ln(y, gk, res, g, b, gf=None, bf=None):
  mspec = pl.BlockSpec((MBLK, D), lambda i: (i, 0))
  gkspec = pl.BlockSpec((MBLK, 1), lambda i: (i, 0))
  bspec = pl.BlockSpec((1, D), lambda i: (0, 0))
  if gf is None:
    body, extra, especs = _combine_ln_body, (), ()
  else:
    body, extra, especs = (_combine_ln_final_body, (gf, bf),
                           (bspec, bspec))
  return pl.pallas_call(
      body,
      grid=(NMB,),
      in_specs=[mspec, gkspec, mspec, bspec, bspec, *especs],
      out_specs=mspec,
      out_shape=jax.ShapeDtypeStruct((T, D), jnp.float32),
      compiler_params=pltpu.CompilerParams(
          dimension_semantics=("arbitrary",)),
  )(y, gk, res, g, b, *extra)


def _loss_body(l1_ref, l2_ref, e1_ref, e2_ref, z_ref, aux_ref):
  zsum = jnp.float32(0.0)
  psum = jnp.zeros((1, E), jnp.float32)
  msum = jnp.zeros((1, E), jnp.float32)
  for l_ref, e_ref in ((l1_ref, e1_ref), (l2_ref, e2_ref)):
    logits = l_ref[...]
    m = jnp.max(logits, axis=-1, keepdims=True)
    ex = jnp.exp(logits - m)
    se = jnp.sum(ex, axis=-1, keepdims=True)
    lse = m + jnp.log(se)
    zsum = zsum + jnp.sum(lse * lse)
    psum = psum + jnp.sum(ex / se, axis=0, keepdims=True)
    iota_e = lax.broadcasted_iota(jnp.int32, (T, E), 1)
    msum = msum + jnp.sum((iota_e == e_ref[...]).astype(jnp.float32),
                          axis=0, keepdims=True)
  n = jnp.float32(L * T)
  z_ref[...] = (zsum / n).reshape(1, 1)
  aux_ref[...] = (jnp.sum((msum / n) * (psum / n)) * ((E * E) / E)).reshape(1, 1)


def _loss(l1, l2, e1, e2):
  spec = lambda shp: pl.BlockSpec(shp, lambda: (0,) * len(shp))
  return pl.pallas_call(
      _loss_body,
      in_specs=[spec((T, E)), spec((T, E)), spec((T, 1)), spec((T, 1))],
      out_specs=[spec((1, 1)), spec((1, 1))],
      out_shape=[jax.ShapeDtypeStruct((1, 1), jnp.float32),
                 jax.ShapeDtypeStruct((1, 1), jnp.float32)],
  )(l1, l2, e1, e2)


def _rope_tables():
  inv = 1.0 / (10000.0 ** (np.arange(0, HD, 2, dtype=np.float64) / HD))
  ang = np.arange(S, dtype=np.float64)[:, None] * inv[None, :]
  sin = np.asarray(np.sin(ang), np.float32)
  cos = np.asarray(np.cos(ang), np.float32)
  sin_f = np.tile(np.tile(sin, (1, H)), (B, 1))
  cos_f = np.tile(np.tile(cos, (1, H)), (B, 1))
  return jnp.asarray(sin_f), jnp.asarray(cos_f)


_EVEN_ODD_PERM = np.concatenate([
    np.arange(D).reshape(H, HD)[:, 0::2].reshape(-1),
    np.arange(D).reshape(H, HD)[:, 1::2].reshape(-1),
])


def kernel(input_ids, params):
  p = params
  ids = input_ids.reshape(T).astype(jnp.int32)
  sin_f, cos_f = _rope_tables()

  x = _sc_gather(p['embed'], ids)

  all_logits = []
  all_eidx = []
  for l in range(L):
    wq_p = p['Wq'][l][:, _EVEN_ODD_PERM]
    wk_p = p['Wk'][l][:, _EVEN_ODD_PERM]
    bq_p = p['bq'][l][_EVEN_ODD_PERM].reshape(1, D)
    bk_p = p['bk'][l][_EVEN_ODD_PERM].reshape(1, D)

    q1, q2, k1, k2, v = _qkv(x, wq_p, wk_p, p['Wv'][l], bq_p, bk_p,
                             p['bv'][l].reshape(1, D), sin_f, cos_f)
    a = _attention(q1, q2, k1, k2, v)
    x = _proj_ln(a, p['Wo'][l], p['bo'][l].reshape(1, D), x,
                 p['ln1_g'][l].reshape(1, D), p['ln1_b'][l].reshape(1, D))

    logits, eidx, gatek, destd, destc = _router(x, p['Wr'][l])
    xe = _sc_scatter(x, destd.reshape(T), T + _SC_NW)
    ye = _ffn(xe[:T].reshape(E, CAP, D), p['W1'][l],
              p['b1'][l].reshape(E, 1, F), p['W2'][l],
              p['b2'][l].reshape(E, 1, D))
    y = _sc_gather(ye.reshape(E * CAP, D), destc.reshape(T))

    if l == L - 1:
      x = _combine_ln(y, gatek, x, p['ln2_g'][l].reshape(1, D),
                      p['ln2_b'][l].reshape(1, D),
                      p['final_g'].reshape(1, D), p['final_b'].reshape(1, D))
    else:
      x = _combine_ln(y, gatek, x, p['ln2_g'][l].reshape(1, D),
                      p['ln2_b'][l].reshape(1, D))
    all_logits.append(logits)
    all_eidx.append(eidx)

  z, aux = _loss(all_logits[0], all_logits[1], all_eidx[0], all_eidx[1])
  return x.reshape(B, S, D), z[0, 0], aux[0, 0]


# speedup vs baseline: 2.0150x; 1.1218x over previous
import functools
import numpy as np
import jax
import jax.numpy as jnp
from jax import lax
from jax.experimental import pallas as pl
from jax.experimental.pallas import tpu as pltpu, tpu_sc as plsc

B = 2
S = 2048
D = 768
F = 3072
L = 2
H = 12
E = 8
CAP = 512
HD = D // H
HF = HD // 2
T = B * S
MBLK = 512
NMB = T // MBLK
_HPG = 4

_SC_NW = 32
_SC_BPW = T // _SC_NW


def _sc_gather(table, idx):
  V, Dd = table.shape
  mesh = plsc.VectorSubcoreMesh(core_axis_name="c", subcore_axis_name="s")

  @functools.partial(
      pl.kernel, mesh=mesh,
      out_type=jax.ShapeDtypeStruct((T, Dd), jnp.float32),
      scratch_types=[
          pltpu.VMEM((_SC_BPW,), jnp.int32),
          pltpu.VMEM((_SC_BPW, Dd), jnp.float32),
          pltpu.SemaphoreType.DMA,
      ],
  )
  def k(table_hbm, idx_hbm, out_hbm, idx_v, rows_v, sem):
    wid = lax.axis_index("s") * 2 + lax.axis_index("c")
    base = wid * _SC_BPW
    pltpu.sync_copy(idx_hbm.at[pl.ds(base, _SC_BPW)], idx_v)
    pltpu.async_copy(table_hbm.at[idx_v], rows_v, sem).wait()
    pltpu.sync_copy(rows_v, out_hbm.at[pl.ds(base, _SC_BPW)])

  return k(table, idx)


def _sc_scatter(rows, dest, nrows):
  Dd = rows.shape[1]
  mesh = plsc.VectorSubcoreMesh(core_axis_name="c", subcore_axis_name="s")

  @functools.partial(
      pl.kernel, mesh=mesh,
      out_type=jax.ShapeDtypeStruct((nrows, Dd), jnp.float32),
      scratch_types=[
          pltpu.VMEM((_SC_BPW,), jnp.int32),
          pltpu.VMEM((_SC_BPW, Dd), jnp.float32),
          pltpu.SemaphoreType.DMA,
      ],
  )
  def k(rows_hbm, dest_hbm, out_hbm, idx_v, rows_v, sem):
    wid = lax.axis_index("s") * 2 + lax.axis_index("c")
    base = wid * _SC_BPW
    pltpu.sync_copy(dest_hbm.at[pl.ds(base, _SC_BPW)], idx_v)
    pltpu.sync_copy(rows_hbm.at[pl.ds(base, _SC_BPW)], rows_v)
    pltpu.async_copy(rows_v, out_hbm.at[idx_v], sem).wait()

  return k(rows, dest)


def _qkv_body(x_ref, wq_ref, wk_ref, wv_ref, bq_ref, bk_ref, bv_ref,
              sin_ref, cos_ref, q_ref, k_ref, v_ref):
  x = x_ref[...]
  s = sin_ref[...]
  c = cos_ref[...]

  q = jnp.dot(x, wq_ref[...], preferred_element_type=jnp.float32) + bq_ref[...]
  q1 = q[:, :D // 2]
  q2 = q[:, D // 2:]
  qr1 = q1 * c - q2 * s
  qr2 = q1 * s + q2 * c

  k = jnp.dot(x, wk_ref[...], preferred_element_type=jnp.float32) + bk_ref[...]
  k1 = k[:, :D // 2]
  k2 = k[:, D // 2:]
  kr1 = k1 * c - k2 * s
  kr2 = k1 * s + k2 * c

  v = jnp.dot(x, wv_ref[...], preferred_element_type=jnp.float32) + bv_ref[...]

  for h in range(H):
    hf = slice(h * HF, (h + 1) * HF)
    q_ref[0, h, :, :HF] = qr1[:, hf]
    q_ref[0, h, :, HF:] = qr2[:, hf]
    k_ref[0, h, :, :HF] = kr1[:, hf]
    k_ref[0, h, :, HF:] = kr2[:, hf]
    v_ref[0, h] = v[:, h * HD:(h + 1) * HD]


def _qkv(x, wq_p, wk_p, wv, bq_p, bk_p, bv, sin_f, cos_f):
  hshape = jax.ShapeDtypeStruct((B, H, S, HD), jnp.float32)
  mspec = lambda n: pl.BlockSpec((MBLK, n), lambda i: (i, 0))
  wspec = pl.BlockSpec((D, D), lambda i: (0, 0))
  bspec = pl.BlockSpec((1, D), lambda i: (0, 0))
  NQB = S // MBLK
  ospec = pl.BlockSpec((1, H, MBLK, HD), lambda i: (i // NQB, 0, i % NQB, 0))
  return pl.pallas_call(
      _qkv_body,
      grid=(NMB,),
      in_specs=[mspec(D), wspec, wspec, wspec, bspec, bspec, bspec,
                mspec(D // 2), mspec(D // 2)],
      out_specs=[ospec, ospec, ospec],
      out_shape=[hshape, hshape, hshape],
      compiler_params=pltpu.CompilerParams(
          dimension_semantics=("arbitrary",)),
  )(x, wq_p, wk_p, wv, bq_p, bk_p, bv, sin_f, cos_f)


def _attn_body(q_ref, k_ref, v_ref, o_ref):
  dn = (((1,), (1,)), ((), ()))
  scale = np.float32(1.0 / np.sqrt(HD))
  for h in range(_HPG):
    s = lax.dot_general(q_ref[0, h], k_ref[0, h], dn,
                        preferred_element_type=jnp.float32)
    s = s * scale
    m = jnp.max(s, axis=-1, keepdims=True)
    p = jnp.exp(s - m)
    l = jnp.sum(p, axis=-1, keepdims=True)
    o = jnp.dot(p, v_ref[0, h], preferred_element_type=jnp.float32)
    o_ref[0, h] = o / l


def _attention(q, k, v):
  QBLK = 512
  nq = S // QBLK
  ng = H // _HPG
  qspec = pl.BlockSpec((1, _HPG, QBLK, HD), lambda b, g, i: (b, g, i, 0))
  kspec = pl.BlockSpec((1, _HPG, S, HD), lambda b, g, i: (b, g, 0, 0))
  ospec = pl.BlockSpec((1, _HPG, QBLK, HD), lambda b, g, i: (b, g, i, 0))
  return pl.pallas_call(
      _attn_body,
      grid=(B, ng, nq),
      in_specs=[qspec, kspec, kspec],
      out_specs=ospec,
      out_shape=jax.ShapeDtypeStruct((B, H, S, HD), jnp.float32),
      compiler_params=pltpu.CompilerParams(
          dimension_semantics=("arbitrary", "arbitrary", "arbitrary")),
  )(q, k, v)


def _ln(t, g, b):
  m = jnp.mean(t, axis=-1, keepdims=True)
  d = t - m
  var = jnp.mean(d * d, axis=-1, keepdims=True)
  return d * lax.rsqrt(var + 1e-5) * g + b


def _proj_ln_body(a_ref, wo_ref, bo_ref, res_ref, g_ref, b_ref, o_ref):
  a = jnp.concatenate([a_ref[0, h] for h in range(H)], axis=-1)
  t = jnp.dot(a, wo_ref[...], preferred_element_type=jnp.float32)
  t = t + bo_ref[...] + res_ref[...]
  o_ref[...] = _ln(t, g_ref[...], b_ref[...])


def _proj_ln(a, wo, bo, res, g, b):
  NQB = S // MBLK
  aspec = pl.BlockSpec((1, H, MBLK, HD), lambda i: (i // NQB, 0, i % NQB, 0))
  mspec = pl.BlockSpec((MBLK, D), lambda i: (i, 0))
  wspec = pl.BlockSpec((D, D), lambda i: (0, 0))
  bspec = pl.BlockSpec((1, D), lambda i: (0, 0))
  return pl.pallas_call(
      _proj_ln_body,
      grid=(NMB,),
      in_specs=[aspec, wspec, bspec, mspec, bspec, bspec],
      out_specs=mspec,
      out_shape=jax.ShapeDtypeStruct((T, D), jnp.float32),
      compiler_params=pltpu.CompilerParams(
          dimension_semantics=("arbitrary",)),
  )(a, wo, bo, res, g, b)


def _router_body(x_ref, wr_ref, logits_ref, eidx_ref, gatek_ref,
                 destd_ref, destc_ref, oh_ref, pos_ref):
  logits = jnp.dot(x_ref[...], wr_ref[...],
                   preferred_element_type=jnp.float32)
  logits_ref[...] = logits
  m = jnp.max(logits, axis=-1, keepdims=True)
  p = jnp.exp(logits - m)
  p = p / jnp.sum(p, axis=-1, keepdims=True)
  gate = jnp.max(p, axis=-1, keepdims=True)
  iota_e = lax.broadcasted_iota(jnp.int32, (T, E), 1)
  eidx = jnp.min(jnp.where(p == gate, iota_e, E), axis=-1,
                 keepdims=True)
  eidx_ref[...] = eidx
  oh_ref[...] = (iota_e == eidx).astype(jnp.float32)

  r = lax.broadcasted_iota(jnp.int32, (MBLK, MBLK), 0)
  cc = lax.broadcasted_iota(jnp.int32, (MBLK, MBLK), 1)
  tril = (r >= cc).astype(jnp.float32)

  def step(j, carry):
    blk = oh_ref[pl.ds(j * MBLK, MBLK), :]
    cum = jnp.dot(tril, blk, preferred_element_type=jnp.float32) + carry
    pos_ref[pl.ds(j * MBLK, MBLK), :] = (
        jnp.sum(cum * blk, axis=-1, keepdims=True) - 1.0)
    return carry + jnp.sum(blk, axis=0, keepdims=True)

  lax.fori_loop(0, NMB, step, jnp.zeros((1, E), jnp.float32))

  pos = pos_ref[...].astype(jnp.int32)
  keep = pos < CAP
  gatek_ref[...] = gate * keep.astype(jnp.float32)
  slot = eidx * CAP + pos
  tok = lax.broadcasted_iota(jnp.int32, (T, 1), 0)
  trash = T + tok // _SC_BPW
  destd_ref[...] = jnp.where(keep, slot, trash)
  destc_ref[...] = eidx * CAP + jnp.minimum(pos, CAP - 1)


def _router(x, wr):
  full = lambda shp, dt: jax.ShapeDtypeStruct(shp, dt)
  spec = lambda shp: pl.BlockSpec(shp, lambda: (0,) * len(shp))
  return pl.pallas_call(
      _router_body,
      in_specs=[spec((T, D)), spec((D, E))],
      out_specs=[spec((T, E)), spec((T, 1)), spec((T, 1)), spec((T, 1)),
                 spec((T, 1))],
      out_shape=[full((T, E), jnp.float32), full((T, 1), jnp.int32),
                 full((T, 1), jnp.float32), full((T, 1), jnp.int32),
                 full((T, 1), jnp.int32)],
      scratch_shapes=[pltpu.VMEM((T, E), jnp.float32),
                      pltpu.VMEM((T, 1), jnp.float32)],
  )(x, wr)


def _erf(x):
  a1, a2, a3, a4, a5 = (0.254829592, -0.284496736, 1.421413741,
                        -1.453152027, 1.061405429)
  sgn = jnp.sign(x)
  ax = jnp.abs(x)
  t = 1.0 / (1.0 + 0.3275911 * ax)
  poly = ((((a5 * t + a4) * t + a3) * t + a2) * t + a1) * t
  return sgn * (1.0 - poly * jnp.exp(-ax * ax))


def _gelu(x):
  return x * 0.5 * (1.0 + lax.erf(x * np.float32(1.0 / np.sqrt(2.0))))


def _ffn_body(xe_ref, w1_ref, b1_ref, w2_ref, b2_ref, ye_ref):
  f = pl.program_id(1)
  h = _gelu(jnp.dot(xe_ref[0], w1_ref[0],
                    preferred_element_type=jnp.float32) + b1_ref[0])
  acc = jnp.dot(h, w2_ref[0], preferred_element_type=jnp.float32)

  @pl.when(f == 0)
  def _():
    ye_ref[0] = acc + b2_ref[0]

  @pl.when(f != 0)
  def _():
    ye_ref[0] = ye_ref[0] + acc


def _ffn(xe, w1, b1, w2, b2):
  FBLK = 768
  nf = F // FBLK
  return pl.pallas_call(
      _ffn_body,
      grid=(E, nf),
      in_specs=[
          pl.BlockSpec((1, CAP, D), lambda e, f: (e, 0, 0)),
          pl.BlockSpec((1, D, FBLK), lambda e, f: (e, 0, f)),
          pl.BlockSpec((1, 1, FBLK), lambda e, f: (e, 0, f)),
          pl.BlockSpec((1, FBLK, D), lambda e, f: (e, f, 0)),
          pl.BlockSpec((1, 1, D), lambda e, f: (e, 0, 0)),
      ],
      out_specs=pl.BlockSpec((1, CAP, D), lambda e, f: (e, 0, 0)),
      out_shape=jax.ShapeDtypeStruct((E, CAP, D), jnp.float32),
      compiler_params=pltpu.CompilerParams(
          dimension_semantics=("arbitrary", "arbitrary")),
  )(xe, w1, b1, w2, b2)


def _combine_ln_body(y_ref, gk_ref, res_ref, g_ref, b_ref, o_ref):
  t = res_ref[...] + y_ref[...] * gk_ref[...]
  o_ref[...] = _ln(t, g_ref[...], b_ref[...])


def _combine_ln_final_body(y_ref, gk_ref, res_ref, g_ref, b_ref,
                           gf_ref, bf_ref, o_ref):
  t = res_ref[...] + y_ref[...] * gk_ref[...]
  t = _ln(t, g_ref[...], b_ref[...])
  o_ref[...] = _ln(t, gf_ref[...], bf_ref[...])


def _combine_ln(y, gk, res, g, b, gf=None, bf=None):
  mspec = pl.BlockSpec((MBLK, D), lambda i: (i, 0))
  gkspec = pl.BlockSpec((MBLK, 1), lambda i: (i, 0))
  bspec = pl.BlockSpec((1, D), lambda i: (0, 0))
  if gf is None:
    body, extra, especs = _combine_ln_body, (), ()
  else:
    body, extra, especs = (_combine_ln_final_body, (gf, bf),
                           (bspec, bspec))
  return pl.pallas_call(
      body,
      grid=(NMB,),
      in_specs=[mspec, gkspec, mspec, bspec, bspec, *especs],
      out_specs=mspec,
      out_shape=jax.ShapeDtypeStruct((T, D), jnp.float32),
      compiler_params=pltpu.CompilerParams(
          dimension_semantics=("arbitrary",)),
  )(y, gk, res, g, b, *extra)


def _loss_body(l1_ref, l2_ref, e1_ref, e2_ref, z_ref, aux_ref):
  zsum = jnp.float32(0.0)
  psum = jnp.zeros((1, E), jnp.float32)
  msum = jnp.zeros((1, E), jnp.float32)
  for l_ref, e_ref in ((l1_ref, e1_ref), (l2_ref, e2_ref)):
    logits = l_ref[...]
    m = jnp.max(logits, axis=-1, keepdims=True)
    ex = jnp.exp(logits - m)
    se = jnp.sum(ex, axis=-1, keepdims=True)
    lse = m + jnp.log(se)
    zsum = zsum + jnp.sum(lse * lse)
    psum = psum + jnp.sum(ex / se, axis=0, keepdims=True)
    iota_e = lax.broadcasted_iota(jnp.int32, (T, E), 1)
    msum = msum + jnp.sum((iota_e == e_ref[...]).astype(jnp.float32),
                          axis=0, keepdims=True)
  n = jnp.float32(L * T)
  z_ref[...] = (zsum / n).reshape(1, 1)
  aux_ref[...] = (jnp.sum((msum / n) * (psum / n)) * ((E * E) / E)).reshape(1, 1)


def _loss(l1, l2, e1, e2):
  spec = lambda shp: pl.BlockSpec(shp, lambda: (0,) * len(shp))
  return pl.pallas_call(
      _loss_body,
      in_specs=[spec((T, E)), spec((T, E)), spec((T, 1)), spec((T, 1))],
      out_specs=[spec((1, 1)), spec((1, 1))],
      out_shape=[jax.ShapeDtypeStruct((1, 1), jnp.float32),
                 jax.ShapeDtypeStruct((1, 1), jnp.float32)],
  )(l1, l2, e1, e2)


def _rope_tables():
  inv = 1.0 / (10000.0 ** (np.arange(0, HD, 2, dtype=np.float64) / HD))
  ang = np.arange(S, dtype=np.float64)[:, None] * inv[None, :]
  sin = np.asarray(np.sin(ang), np.float32)
  cos = np.asarray(np.cos(ang), np.float32)
  sin_f = np.tile(np.tile(sin, (1, H)), (B, 1))
  cos_f = np.tile(np.tile(cos, (1, H)), (B, 1))
  return jnp.asarray(sin_f), jnp.asarray(cos_f)


_EVEN_ODD_PERM = np.concatenate([
    np.arange(D).reshape(H, HD)[:, 0::2].reshape(-1),
    np.arange(D).reshape(H, HD)[:, 1::2].reshape(-1),
])


def kernel(input_ids, params):
  p = params
  ids = input_ids.reshape(T).astype(jnp.int32)
  sin_f, cos_f = _rope_tables()

  x = _sc_gather(p['embed'], ids)

  all_logits = []
  all_eidx = []
  for l in range(L):
    wq_p = p['Wq'][l][:, _EVEN_ODD_PERM]
    wk_p = p['Wk'][l][:, _EVEN_ODD_PERM]
    bq_p = p['bq'][l][_EVEN_ODD_PERM].reshape(1, D)
    bk_p = p['bk'][l][_EVEN_ODD_PERM].reshape(1, D)

    q, k, v = _qkv(x, wq_p, wk_p, p['Wv'][l], bq_p, bk_p,
                   p['bv'][l].reshape(1, D), sin_f, cos_f)
    a = _attention(q, k, v)
    x = _proj_ln(a, p['Wo'][l], p['bo'][l].reshape(1, D), x,
                 p['ln1_g'][l].reshape(1, D), p['ln1_b'][l].reshape(1, D))

    logits, eidx, gatek, destd, destc = _router(x, p['Wr'][l])
    xe = _sc_scatter(x, destd.reshape(T), T + _SC_NW)
    ye = _ffn(xe[:T].reshape(E, CAP, D), p['W1'][l],
              p['b1'][l].reshape(E, 1, F), p['W2'][l],
              p['b2'][l].reshape(E, 1, D))
    y = _sc_gather(ye.reshape(E * CAP, D), destc.reshape(T))

    if l == L - 1:
      x = _combine_ln(y, gatek, x, p['ln2_g'][l].reshape(1, D),
                      p['ln2_b'][l].reshape(1, D),
                      p['final_g'].reshape(1, D), p['final_b'].reshape(1, D))
    else:
      x = _combine_ln(y, gatek, x, p['ln2_g'][l].reshape(1, D),
                      p['ln2_b'][l].reshape(1, D))
    all_logits.append(logits)
    all_eidx.append(eidx)

  z, aux = _loss(all_logits[0], all_logits[1], all_eidx[0], all_eidx[1])
  return x.reshape(B, S, D), z[0, 0], aux[0, 0]


# speedup vs baseline: 2.2416x; 1.1125x over previous
import functools
import numpy as np
import jax
import jax.numpy as jnp
from jax import lax
from jax.experimental import pallas as pl
from jax.experimental.pallas import tpu as pltpu, tpu_sc as plsc

B = 2
S = 2048
D = 768
F = 3072
L = 2
H = 12
E = 8
CAP = 512
HD = D // H
HF = HD // 2
T = B * S
MBLK = 512
NMB = T // MBLK
_HPG = 4

_SC_NW = 32
_SC_BPW = T // _SC_NW


def _sc_gather(table, idx):
  V, Dd = table.shape
  mesh = plsc.VectorSubcoreMesh(core_axis_name="c", subcore_axis_name="s")

  @functools.partial(
      pl.kernel, mesh=mesh,
      out_type=jax.ShapeDtypeStruct((T, Dd), jnp.float32),
      scratch_types=[
          pltpu.VMEM((_SC_BPW,), jnp.int32),
          pltpu.VMEM((_SC_BPW, Dd), jnp.float32),
          pltpu.SemaphoreType.DMA,
      ],
  )
  def k(table_hbm, idx_hbm, out_hbm, idx_v, rows_v, sem):
    wid = lax.axis_index("s") * 2 + lax.axis_index("c")
    base = wid * _SC_BPW
    pltpu.sync_copy(idx_hbm.at[pl.ds(base, _SC_BPW)], idx_v)
    pltpu.async_copy(table_hbm.at[idx_v], rows_v, sem).wait()
    pltpu.sync_copy(rows_v, out_hbm.at[pl.ds(base, _SC_BPW)])

  return k(table, idx)


def _sc_scatter(rows, dest, nrows):
  Dd = rows.shape[1]
  mesh = plsc.VectorSubcoreMesh(core_axis_name="c", subcore_axis_name="s")

  @functools.partial(
      pl.kernel, mesh=mesh,
      out_type=jax.ShapeDtypeStruct((nrows, Dd), jnp.float32),
      scratch_types=[
          pltpu.VMEM((_SC_BPW,), jnp.int32),
          pltpu.VMEM((_SC_BPW, Dd), jnp.float32),
          pltpu.SemaphoreType.DMA,
      ],
  )
  def k(rows_hbm, dest_hbm, out_hbm, idx_v, rows_v, sem):
    wid = lax.axis_index("s") * 2 + lax.axis_index("c")
    base = wid * _SC_BPW
    pltpu.sync_copy(dest_hbm.at[pl.ds(base, _SC_BPW)], idx_v)
    pltpu.sync_copy(rows_hbm.at[pl.ds(base, _SC_BPW)], rows_v)
    pltpu.async_copy(rows_v, out_hbm.at[idx_v], sem).wait()

  return k(rows, dest)


def _qkv_body(x_ref, wq_ref, wk_ref, wv_ref, bq_ref, bk_ref, bv_ref,
              sin_ref, cos_ref, q_ref, k_ref, v_ref):
  x = x_ref[...]
  s = sin_ref[...]
  c = cos_ref[...]

  q = jnp.dot(x, wq_ref[...], preferred_element_type=jnp.float32) + bq_ref[...]
  q1 = q[:, :D // 2]
  q2 = q[:, D // 2:]
  qr1 = q1 * c - q2 * s
  qr2 = q1 * s + q2 * c

  k = jnp.dot(x, wk_ref[...], preferred_element_type=jnp.float32) + bk_ref[...]
  k1 = k[:, :D // 2]
  k2 = k[:, D // 2:]
  kr1 = k1 * c - k2 * s
  kr2 = k1 * s + k2 * c

  v = jnp.dot(x, wv_ref[...], preferred_element_type=jnp.float32) + bv_ref[...]

  scale = np.float32(1.0 / np.sqrt(HD))
  ones = jnp.ones((MBLK, 1), jnp.float32)
  for h in range(H):
    hf = slice(h * HF, (h + 1) * HF)
    q_ref[0, h, :, :HF] = qr1[:, hf] * scale
    q_ref[0, h, :, HF:] = qr2[:, hf] * scale
    k_ref[0, h, :, :HF] = kr1[:, hf]
    k_ref[0, h, :, HF:] = kr2[:, hf]
    v_ref[0, h, :, :HD] = v[:, h * HD:(h + 1) * HD]
    v_ref[0, h, :, HD:HD + 1] = ones


def _qkv(x, wq_p, wk_p, wv, bq_p, bk_p, bv, sin_f, cos_f):
  hshape = jax.ShapeDtypeStruct((B, H, S, HD), jnp.float32)
  vshape = jax.ShapeDtypeStruct((B, H, S, 2 * HD), jnp.float32)
  mspec = lambda n: pl.BlockSpec((MBLK, n), lambda i: (i, 0))
  wspec = pl.BlockSpec((D, D), lambda i: (0, 0))
  bspec = pl.BlockSpec((1, D), lambda i: (0, 0))
  NQB = S // MBLK
  ospec = pl.BlockSpec((1, H, MBLK, HD), lambda i: (i // NQB, 0, i % NQB, 0))
  vspec = pl.BlockSpec((1, H, MBLK, 2 * HD),
                       lambda i: (i // NQB, 0, i % NQB, 0))
  return pl.pallas_call(
      _qkv_body,
      grid=(NMB,),
      in_specs=[mspec(D), wspec, wspec, wspec, bspec, bspec, bspec,
                mspec(D // 2), mspec(D // 2)],
      out_specs=[ospec, ospec, vspec],
      out_shape=[hshape, hshape, vshape],
      compiler_params=pltpu.CompilerParams(
          dimension_semantics=("arbitrary",)),
  )(x, wq_p, wk_p, wv, bq_p, bk_p, bv, sin_f, cos_f)


def _attn_body(q_ref, k_ref, v_ref, o_ref):
  dn = (((1,), (1,)), ((), ()))
  for h in range(_HPG):
    s = lax.dot_general(q_ref[0, h], k_ref[0, h], dn,
                        preferred_element_type=jnp.float32)
    p = jnp.exp(s)
    ol = jnp.dot(p, v_ref[0, h], preferred_element_type=jnp.float32)
    o_ref[0, h] = ol[:, :HD] / ol[:, HD:HD + 1]


def _attention(q, k, v):
  QBLK = 512
  nq = S // QBLK
  ng = H // _HPG
  qspec = pl.BlockSpec((1, _HPG, QBLK, HD), lambda b, g, i: (b, g, i, 0))
  kspec = pl.BlockSpec((1, _HPG, S, HD), lambda b, g, i: (b, g, 0, 0))
  vspec = pl.BlockSpec((1, _HPG, S, 2 * HD), lambda b, g, i: (b, g, 0, 0))
  ospec = pl.BlockSpec((1, _HPG, QBLK, HD), lambda b, g, i: (b, g, i, 0))
  return pl.pallas_call(
      _attn_body,
      grid=(B, ng, nq),
      in_specs=[qspec, kspec, vspec],
      out_specs=ospec,
      out_shape=jax.ShapeDtypeStruct((B, H, S, HD), jnp.float32),
      compiler_params=pltpu.CompilerParams(
          dimension_semantics=("arbitrary", "arbitrary", "arbitrary")),
  )(q, k, v)


def _ln(t, g, b):
  m = jnp.mean(t, axis=-1, keepdims=True)
  d = t - m
  var = jnp.mean(d * d, axis=-1, keepdims=True)
  return d * lax.rsqrt(var + 1e-5) * g + b


def _proj_ln_body(a_ref, wo_ref, bo_ref, res_ref, g_ref, b_ref, o_ref):
  a = jnp.concatenate([a_ref[0, h] for h in range(H)], axis=-1)
  t = jnp.dot(a, wo_ref[...], preferred_element_type=jnp.float32)
  t = t + bo_ref[...] + res_ref[...]
  o_ref[...] = _ln(t, g_ref[...], b_ref[...])


def _proj_ln(a, wo, bo, res, g, b):
  NQB = S // MBLK
  aspec = pl.BlockSpec((1, H, MBLK, HD), lambda i: (i // NQB, 0, i % NQB, 0))
  mspec = pl.BlockSpec((MBLK, D), lambda i: (i, 0))
  wspec = pl.BlockSpec((D, D), lambda i: (0, 0))
  bspec = pl.BlockSpec((1, D), lambda i: (0, 0))
  return pl.pallas_call(
      _proj_ln_body,
      grid=(NMB,),
      in_specs=[aspec, wspec, bspec, mspec, bspec, bspec],
      out_specs=mspec,
      out_shape=jax.ShapeDtypeStruct((T, D), jnp.float32),
      compiler_params=pltpu.CompilerParams(
          dimension_semantics=("arbitrary",)),
  )(a, wo, bo, res, g, b)


def _router_body(x_ref, wr_ref, logits_ref, eidx_ref, gatek_ref,
                 destd_ref, destc_ref, oh_ref, pos_ref):
  logits = jnp.dot(x_ref[...], wr_ref[...],
                   preferred_element_type=jnp.float32)
  logits_ref[...] = logits
  m = jnp.max(logits, axis=-1, keepdims=True)
  p = jnp.exp(logits - m)
  p = p / jnp.sum(p, axis=-1, keepdims=True)
  gate = jnp.max(p, axis=-1, keepdims=True)
  iota_e = lax.broadcasted_iota(jnp.int32, (T, E), 1)
  eidx = jnp.min(jnp.where(p == gate, iota_e, E), axis=-1,
                 keepdims=True)
  eidx_ref[...] = eidx
  oh_ref[...] = (iota_e == eidx).astype(jnp.float32)

  r = lax.broadcasted_iota(jnp.int32, (MBLK, MBLK), 0)
  cc = lax.broadcasted_iota(jnp.int32, (MBLK, MBLK), 1)
  tril = (r >= cc).astype(jnp.float32)

  def step(j, carry):
    blk = oh_ref[pl.ds(j * MBLK, MBLK), :]
    cum = jnp.dot(tril, blk, preferred_element_type=jnp.float32) + carry
    pos_ref[pl.ds(j * MBLK, MBLK), :] = (
        jnp.sum(cum * blk, axis=-1, keepdims=True) - 1.0)
    return carry + jnp.sum(blk, axis=0, keepdims=True)

  lax.fori_loop(0, NMB, step, jnp.zeros((1, E), jnp.float32))

  pos = pos_ref[...].astype(jnp.int32)
  keep = pos < CAP
  gatek_ref[...] = gate * keep.astype(jnp.float32)
  slot = eidx * CAP + pos
  tok = lax.broadcasted_iota(jnp.int32, (T, 1), 0)
  trash = T + tok // _SC_BPW
  destd_ref[...] = jnp.where(keep, slot, trash)
  destc_ref[...] = eidx * CAP + jnp.minimum(pos, CAP - 1)


def _router(x, wr):
  full = lambda shp, dt: jax.ShapeDtypeStruct(shp, dt)
  spec = lambda shp: pl.BlockSpec(shp, lambda: (0,) * len(shp))
  return pl.pallas_call(
      _router_body,
      in_specs=[spec((T, D)), spec((D, E))],
      out_specs=[spec((T, E)), spec((T, 1)), spec((T, 1)), spec((T, 1)),
                 spec((T, 1))],
      out_shape=[full((T, E), jnp.float32), full((T, 1), jnp.int32),
                 full((T, 1), jnp.float32), full((T, 1), jnp.int32),
                 full((T, 1), jnp.int32)],
      scratch_shapes=[pltpu.VMEM((T, E), jnp.float32),
                      pltpu.VMEM((T, 1), jnp.float32)],
  )(x, wr)


def _erf(x):
  a1, a2, a3, a4, a5 = (0.254829592, -0.284496736, 1.421413741,
                        -1.453152027, 1.061405429)
  sgn = jnp.sign(x)
  ax = jnp.abs(x)
  t = 1.0 / (1.0 + 0.3275911 * ax)
  poly = ((((a5 * t + a4) * t + a3) * t + a2) * t + a1) * t
  return sgn * (1.0 - poly * jnp.exp(-ax * ax))


def _gelu(x):
  return x * 0.5 * (1.0 + lax.erf(x * np.float32(1.0 / np.sqrt(2.0))))


def _ffn_body(xe_ref, w1_ref, b1_ref, w2_ref, b2_ref, ye_ref):
  f = pl.program_id(1)
  h = _gelu(jnp.dot(xe_ref[0], w1_ref[0],
                    preferred_element_type=jnp.float32) + b1_ref[0])
  acc = jnp.dot(h, w2_ref[0], preferred_element_type=jnp.float32)

  @pl.when(f == 0)
  def _():
    ye_ref[0] = acc + b2_ref[0]

  @pl.when(f != 0)
  def _():
    ye_ref[0] = ye_ref[0] + acc


def _ffn(xe, w1, b1, w2, b2):
  FBLK = 768
  nf = F // FBLK
  return pl.pallas_call(
      _ffn_body,
      grid=(E, nf),
      in_specs=[
          pl.BlockSpec((1, CAP, D), lambda e, f: (e, 0, 0)),
          pl.BlockSpec((1, D, FBLK), lambda e, f: (e, 0, f)),
          pl.BlockSpec((1, 1, FBLK), lambda e, f: (e, 0, f)),
          pl.BlockSpec((1, FBLK, D), lambda e, f: (e, f, 0)),
          pl.BlockSpec((1, 1, D), lambda e, f: (e, 0, 0)),
      ],
      out_specs=pl.BlockSpec((1, CAP, D), lambda e, f: (e, 0, 0)),
      out_shape=jax.ShapeDtypeStruct((E, CAP, D), jnp.float32),
      compiler_params=pltpu.CompilerParams(
          dimension_semantics=("arbitrary", "arbitrary")),
  )(xe, w1, b1, w2, b2)


def _combine_ln_body(y_ref, gk_ref, res_ref, g_ref, b_ref, o_ref):
  t = res_ref[...] + y_ref[...] * gk_ref[...]
  o_ref[...] = _ln(t, g_ref[...], b_ref[...])


def _combine_ln_final_body(y_ref, gk_ref, res_ref, g_ref, b_ref,
                           gf_ref, bf_ref, o_ref):
  t = res_ref[...] + y_ref[...] * gk_ref[...]
  t = _ln(t, g_ref[...], b_ref[...])
  o_ref[...] = _ln(t, gf_ref[...], bf_ref[...])


def _combine_ln(y, gk, res, g, b, gf=None, bf=None):
  mspec = pl.BlockSpec((MBLK, D), lambda i: (i, 0))
  gkspec = pl.BlockSpec((MBLK, 1), lambda i: (i, 0))
  bspec = pl.BlockSpec((1, D), lambda i: (0, 0))
  if gf is None:
    body, extra, especs = _combine_ln_body, (), ()
  else:
    body, extra, especs = (_combine_ln_final_body, (gf, bf),
                           (bspec, bspec))
  return pl.pallas_call(
      body,
      grid=(NMB,),
      in_specs=[mspec, gkspec, mspec, bspec, bspec, *especs],
      out_specs=mspec,
      out_shape=jax.ShapeDtypeStruct((T, D), jnp.float32),
      compiler_params=pltpu.CompilerParams(
          dimension_semantics=("arbitrary",)),
  )(y, gk, res, g, b, *extra)


def _loss_body(l1_ref, l2_ref, e1_ref, e2_ref, z_ref, aux_ref):
  zsum = jnp.float32(0.0)
  psum = jnp.zeros((1, E), jnp.float32)
  msum = jnp.zeros((1, E), jnp.float32)
  for l_ref, e_ref in ((l1_ref, e1_ref), (l2_ref, e2_ref)):
    logits = l_ref[...]
    m = jnp.max(logits, axis=-1, keepdims=True)
    ex = jnp.exp(logits - m)
    se = jnp.sum(ex, axis=-1, keepdims=True)
    lse = m + jnp.log(se)
    zsum = zsum + jnp.sum(lse * lse)
    psum = psum + jnp.sum(ex / se, axis=0, keepdims=True)
    iota_e = lax.broadcasted_iota(jnp.int32, (T, E), 1)
    msum = msum + jnp.sum((iota_e == e_ref[...]).astype(jnp.float32),
                          axis=0, keepdims=True)
  n = jnp.float32(L * T)
  z_ref[...] = (zsum / n).reshape(1, 1)
  aux_ref[...] = (jnp.sum((msum / n) * (psum / n)) * ((E * E) / E)).reshape(1, 1)


def _loss(l1, l2, e1, e2):
  spec = lambda shp: pl.BlockSpec(shp, lambda: (0,) * len(shp))
  return pl.pallas_call(
      _loss_body,
      in_specs=[spec((T, E)), spec((T, E)), spec((T, 1)), spec((T, 1))],
      out_specs=[spec((1, 1)), spec((1, 1))],
      out_shape=[jax.ShapeDtypeStruct((1, 1), jnp.float32),
                 jax.ShapeDtypeStruct((1, 1), jnp.float32)],
  )(l1, l2, e1, e2)


def _rope_tables():
  inv = 1.0 / (10000.0 ** (np.arange(0, HD, 2, dtype=np.float64) / HD))
  ang = np.arange(S, dtype=np.float64)[:, None] * inv[None, :]
  sin = np.asarray(np.sin(ang), np.float32)
  cos = np.asarray(np.cos(ang), np.float32)
  sin_f = np.tile(np.tile(sin, (1, H)), (B, 1))
  cos_f = np.tile(np.tile(cos, (1, H)), (B, 1))
  return jnp.asarray(sin_f), jnp.asarray(cos_f)


_EVEN_ODD_PERM = np.concatenate([
    np.arange(D).reshape(H, HD)[:, 0::2].reshape(-1),
    np.arange(D).reshape(H, HD)[:, 1::2].reshape(-1),
])


def kernel(input_ids, params):
  p = params
  ids = input_ids.reshape(T).astype(jnp.int32)
  sin_f, cos_f = _rope_tables()

  x = _sc_gather(p['embed'], ids)

  all_logits = []
  all_eidx = []
  for l in range(L):
    wq_p = p['Wq'][l][:, _EVEN_ODD_PERM]
    wk_p = p['Wk'][l][:, _EVEN_ODD_PERM]
    bq_p = p['bq'][l][_EVEN_ODD_PERM].reshape(1, D)
    bk_p = p['bk'][l][_EVEN_ODD_PERM].reshape(1, D)

    q, k, v = _qkv(x, wq_p, wk_p, p['Wv'][l], bq_p, bk_p,
                   p['bv'][l].reshape(1, D), sin_f, cos_f)
    a = _attention(q, k, v)
    x = _proj_ln(a, p['Wo'][l], p['bo'][l].reshape(1, D), x,
                 p['ln1_g'][l].reshape(1, D), p['ln1_b'][l].reshape(1, D))

    logits, eidx, gatek, destd, destc = _router(x, p['Wr'][l])
    xe = _sc_scatter(x, destd.reshape(T), T + _SC_NW)
    ye = _ffn(xe[:T].reshape(E, CAP, D), p['W1'][l],
              p['b1'][l].reshape(E, 1, F), p['W2'][l],
              p['b2'][l].reshape(E, 1, D))
    y = _sc_gather(ye.reshape(E * CAP, D), destc.reshape(T))

    if l == L - 1:
      x = _combine_ln(y, gatek, x, p['ln2_g'][l].reshape(1, D),
                      p['ln2_b'][l].reshape(1, D),
                      p['final_g'].reshape(1, D), p['final_b'].reshape(1, D))
    else:
      x = _combine_ln(y, gatek, x, p['ln2_g'][l].reshape(1, D),
                      p['ln2_b'][l].reshape(1, D))
    all_logits.append(logits)
    all_eidx.append(eidx)

  z, aux = _loss(all_logits[0], all_logits[1], all_eidx[0], all_eidx[1])
  return x.reshape(B, S, D), z[0, 0], aux[0, 0]


# speedup vs baseline: 2.3171x; 1.0337x over previous
import functools
import numpy as np
import jax
import jax.numpy as jnp
from jax import lax
from jax.experimental import pallas as pl
from jax.experimental.pallas import tpu as pltpu, tpu_sc as plsc

B = 2
S = 2048
D = 768
F = 3072
L = 2
H = 12
E = 8
CAP = 512
HD = D // H
HF = HD // 2
T = B * S
MBLK = 512
NMB = T // MBLK
_HPG = 4

_SC_NW = 32
_SC_BPW = T // _SC_NW


def _sc_gather(table, idx):
  V, Dd = table.shape
  mesh = plsc.VectorSubcoreMesh(core_axis_name="c", subcore_axis_name="s")

  @functools.partial(
      pl.kernel, mesh=mesh,
      out_type=jax.ShapeDtypeStruct((T, Dd), jnp.float32),
      scratch_types=[
          pltpu.VMEM((_SC_BPW,), jnp.int32),
          pltpu.VMEM((_SC_BPW, Dd), jnp.float32),
          pltpu.SemaphoreType.DMA,
      ],
  )
  def k(table_hbm, idx_hbm, out_hbm, idx_v, rows_v, sem):
    wid = lax.axis_index("s") * 2 + lax.axis_index("c")
    base = wid * _SC_BPW
    pltpu.sync_copy(idx_hbm.at[pl.ds(base, _SC_BPW)], idx_v)
    pltpu.async_copy(table_hbm.at[idx_v], rows_v, sem).wait()
    pltpu.sync_copy(rows_v, out_hbm.at[pl.ds(base, _SC_BPW)])

  return k(table, idx)


def _sc_scatter(rows, dest, nrows):
  Dd = rows.shape[1]
  mesh = plsc.VectorSubcoreMesh(core_axis_name="c", subcore_axis_name="s")

  @functools.partial(
      pl.kernel, mesh=mesh,
      out_type=jax.ShapeDtypeStruct((nrows, Dd), jnp.float32),
      scratch_types=[
          pltpu.VMEM((_SC_BPW,), jnp.int32),
          pltpu.VMEM((_SC_BPW, Dd), jnp.float32),
          pltpu.SemaphoreType.DMA,
      ],
  )
  def k(rows_hbm, dest_hbm, out_hbm, idx_v, rows_v, sem):
    wid = lax.axis_index("s") * 2 + lax.axis_index("c")
    base = wid * _SC_BPW
    pltpu.sync_copy(dest_hbm.at[pl.ds(base, _SC_BPW)], idx_v)
    pltpu.sync_copy(rows_hbm.at[pl.ds(base, _SC_BPW)], rows_v)
    pltpu.async_copy(rows_v, out_hbm.at[idx_v], sem).wait()

  return k(rows, dest)


def _qkv_compute(x, wq_ref, wk_ref, wv_ref, bq_ref, bk_ref, bv_ref,
                 sin_ref, cos_ref, q_ref, k_ref, v_ref):
  s = sin_ref[...]
  c = cos_ref[...]

  q = jnp.dot(x, wq_ref[...], preferred_element_type=jnp.float32) + bq_ref[...]
  q1 = q[:, :D // 2]
  q2 = q[:, D // 2:]
  qr1 = q1 * c - q2 * s
  qr2 = q1 * s + q2 * c

  k = jnp.dot(x, wk_ref[...], preferred_element_type=jnp.float32) + bk_ref[...]
  k1 = k[:, :D // 2]
  k2 = k[:, D // 2:]
  kr1 = k1 * c - k2 * s
  kr2 = k1 * s + k2 * c

  v = jnp.dot(x, wv_ref[...], preferred_element_type=jnp.float32) + bv_ref[...]

  scale = np.float32(1.0 / np.sqrt(HD))
  ones = jnp.ones((MBLK, 1), jnp.float32)
  for h in range(H):
    hf = slice(h * HF, (h + 1) * HF)
    q_ref[0, h, :, :HF] = qr1[:, hf] * scale
    q_ref[0, h, :, HF:] = qr2[:, hf] * scale
    k_ref[0, h, :, :HF] = kr1[:, hf]
    k_ref[0, h, :, HF:] = kr2[:, hf]
    v_ref[0, h, :, :HD] = v[:, h * HD:(h + 1) * HD]
    v_ref[0, h, :, HD:HD + 1] = ones


def _qkv_body(x_ref, wq_ref, wk_ref, wv_ref, bq_ref, bk_ref, bv_ref,
              sin_ref, cos_ref, q_ref, k_ref, v_ref):
  _qkv_compute(x_ref[...], wq_ref, wk_ref, wv_ref, bq_ref, bk_ref, bv_ref,
               sin_ref, cos_ref, q_ref, k_ref, v_ref)


def _qkv(x, wq_p, wk_p, wv, bq_p, bk_p, bv, sin_f, cos_f):
  hshape = jax.ShapeDtypeStruct((B, H, S, HD), jnp.float32)
  vshape = jax.ShapeDtypeStruct((B, H, S, 2 * HD), jnp.float32)
  mspec = lambda n: pl.BlockSpec((MBLK, n), lambda i: (i, 0))
  wspec = pl.BlockSpec((D, D), lambda i: (0, 0))
  bspec = pl.BlockSpec((1, D), lambda i: (0, 0))
  NQB = S // MBLK
  ospec = pl.BlockSpec((1, H, MBLK, HD), lambda i: (i // NQB, 0, i % NQB, 0))
  vspec = pl.BlockSpec((1, H, MBLK, 2 * HD),
                       lambda i: (i // NQB, 0, i % NQB, 0))
  return pl.pallas_call(
      _qkv_body,
      grid=(NMB,),
      in_specs=[mspec(D), wspec, wspec, wspec, bspec, bspec, bspec,
                mspec(D // 2), mspec(D // 2)],
      out_specs=[ospec, ospec, vspec],
      out_shape=[hshape, hshape, vshape],
      compiler_params=pltpu.CompilerParams(
          dimension_semantics=("arbitrary",)),
  )(x, wq_p, wk_p, wv, bq_p, bk_p, bv, sin_f, cos_f)


def _attn_body(q_ref, k_ref, v_ref, o_ref):
  dn = (((1,), (1,)), ((), ()))
  for h in range(_HPG):
    s = lax.dot_general(q_ref[0, h], k_ref[0, h], dn,
                        preferred_element_type=jnp.float32)
    p = jnp.exp(s)
    ol = jnp.dot(p, v_ref[0, h], preferred_element_type=jnp.float32)
    o_ref[0, h] = ol[:, :HD] / ol[:, HD:HD + 1]


def _attention(q, k, v):
  QBLK = 512
  nq = S // QBLK
  ng = H // _HPG
  qspec = pl.BlockSpec((1, _HPG, QBLK, HD), lambda b, g, i: (b, g, i, 0))
  kspec = pl.BlockSpec((1, _HPG, S, HD), lambda b, g, i: (b, g, 0, 0))
  vspec = pl.BlockSpec((1, _HPG, S, 2 * HD), lambda b, g, i: (b, g, 0, 0))
  ospec = pl.BlockSpec((1, _HPG, QBLK, HD), lambda b, g, i: (b, g, i, 0))
  return pl.pallas_call(
      _attn_body,
      grid=(B, ng, nq),
      in_specs=[qspec, kspec, vspec],
      out_specs=ospec,
      out_shape=jax.ShapeDtypeStruct((B, H, S, HD), jnp.float32),
      compiler_params=pltpu.CompilerParams(
          dimension_semantics=("arbitrary", "arbitrary", "arbitrary")),
  )(q, k, v)


def _ln(t, g, b):
  m = jnp.mean(t, axis=-1, keepdims=True)
  d = t - m
  var = jnp.mean(d * d, axis=-1, keepdims=True)
  return d * lax.rsqrt(var + 1e-5) * g + b


def _proj_router_body(a_ref, wo_ref, bo_ref, res_ref, g_ref, b_ref, wr_ref,
                      x1_ref, logits_ref, eidx_ref, gatek_ref,
                      destd_ref, destc_ref, carry_ref):
  i = pl.program_id(0)

  @pl.when(i == 0)
  def _():
    carry_ref[...] = jnp.zeros((1, E), jnp.float32)

  a = jnp.concatenate([a_ref[0, h] for h in range(H)], axis=-1)
  t = jnp.dot(a, wo_ref[...], preferred_element_type=jnp.float32)
  t = t + bo_ref[...] + res_ref[...]
  x1 = _ln(t, g_ref[...], b_ref[...])
  x1_ref[...] = x1

  logits = jnp.dot(x1, wr_ref[...], preferred_element_type=jnp.float32)
  logits_ref[...] = logits
  m = jnp.max(logits, axis=-1, keepdims=True)
  p = jnp.exp(logits - m)
  p = p / jnp.sum(p, axis=-1, keepdims=True)
  gate = jnp.max(p, axis=-1, keepdims=True)
  iota_e = lax.broadcasted_iota(jnp.int32, (MBLK, E), 1)
  eidx = jnp.min(jnp.where(p == gate, iota_e, E), axis=-1,
                 keepdims=True)
  eidx_ref[...] = eidx
  oh = (iota_e == eidx).astype(jnp.float32)

  r = lax.broadcasted_iota(jnp.int32, (MBLK, MBLK), 0)
  cc = lax.broadcasted_iota(jnp.int32, (MBLK, MBLK), 1)
  tril = (r >= cc).astype(jnp.float32)
  cum = jnp.dot(tril, oh, preferred_element_type=jnp.float32) + carry_ref[...]
  carry_ref[...] = carry_ref[...] + jnp.sum(oh, axis=0, keepdims=True)

  pos = (jnp.sum(cum * oh, axis=-1, keepdims=True) - 1.0).astype(jnp.int32)
  keep = pos < CAP
  gatek_ref[...] = gate * keep.astype(jnp.float32)
  slot = eidx * CAP + pos
  tok = i * MBLK + lax.broadcasted_iota(jnp.int32, (MBLK, 1), 0)
  trash = T + tok // _SC_BPW
  destd_ref[...] = jnp.where(keep, slot, trash)
  destc_ref[...] = eidx * CAP + jnp.minimum(pos, CAP - 1)


def _proj_router(a, wo, bo, res, g, b, wr):
  NQB = S // MBLK
  aspec = pl.BlockSpec((1, H, MBLK, HD), lambda i: (i // NQB, 0, i % NQB, 0))
  mspec = pl.BlockSpec((MBLK, D), lambda i: (i, 0))
  espec = pl.BlockSpec((MBLK, E), lambda i: (i, 0))
  sspec = pl.BlockSpec((MBLK, 1), lambda i: (i, 0))
  wspec = pl.BlockSpec((D, D), lambda i: (0, 0))
  bspec = pl.BlockSpec((1, D), lambda i: (0, 0))
  rspec = pl.BlockSpec((D, E), lambda i: (0, 0))
  full = lambda shp, dt: jax.ShapeDtypeStruct(shp, dt)
  return pl.pallas_call(
      _proj_router_body,
      grid=(NMB,),
      in_specs=[aspec, wspec, bspec, mspec, bspec, bspec, rspec],
      out_specs=[mspec, espec, sspec, sspec, sspec, sspec],
      out_shape=[full((T, D), jnp.float32), full((T, E), jnp.float32),
                 full((T, 1), jnp.int32), full((T, 1), jnp.float32),
                 full((T, 1), jnp.int32), full((T, 1), jnp.int32)],
      scratch_shapes=[pltpu.VMEM((1, E), jnp.float32)],
      compiler_params=pltpu.CompilerParams(
          dimension_semantics=("arbitrary",)),
  )(a, wo, bo, res, g, b, wr)


def _gelu(x):
  return x * 0.5 * (1.0 + lax.erf(x * np.float32(1.0 / np.sqrt(2.0))))


def _ffn_body(xe_ref, w1_ref, b1_ref, w2_ref, b2_ref, ye_ref):
  f = pl.program_id(1)
  h = _gelu(jnp.dot(xe_ref[0], w1_ref[0],
                    preferred_element_type=jnp.float32) + b1_ref[0])
  acc = jnp.dot(h, w2_ref[0], preferred_element_type=jnp.float32)

  @pl.when(f == 0)
  def _():
    ye_ref[0] = acc + b2_ref[0]

  @pl.when(f != 0)
  def _():
    ye_ref[0] = ye_ref[0] + acc


def _ffn(xe, w1, b1, w2, b2):
  FBLK = 768
  nf = F // FBLK
  return pl.pallas_call(
      _ffn_body,
      grid=(E, nf),
      in_specs=[
          pl.BlockSpec((1, CAP, D), lambda e, f: (e, 0, 0)),
          pl.BlockSpec((1, D, FBLK), lambda e, f: (e, 0, f)),
          pl.BlockSpec((1, 1, FBLK), lambda e, f: (e, 0, f)),
          pl.BlockSpec((1, FBLK, D), lambda e, f: (e, f, 0)),
          pl.BlockSpec((1, 1, D), lambda e, f: (e, 0, 0)),
      ],
      out_specs=pl.BlockSpec((1, CAP, D), lambda e, f: (e, 0, 0)),
      out_shape=jax.ShapeDtypeStruct((E, CAP, D), jnp.float32),
      compiler_params=pltpu.CompilerParams(
          dimension_semantics=("arbitrary", "arbitrary")),
  )(xe, w1, b1, w2, b2)


def _combine_qkv_body(y_ref, gk_ref, res_ref, g_ref, b_ref,
                      wq_ref, wk_ref, wv_ref, bq_ref, bk_ref, bv_ref,
                      sin_ref, cos_ref, x_ref, q_ref, k_ref, v_ref):
  t = res_ref[...] + y_ref[...] * gk_ref[...]
  x = _ln(t, g_ref[...], b_ref[...])
  x_ref[...] = x
  _qkv_compute(x, wq_ref, wk_ref, wv_ref, bq_ref, bk_ref, bv_ref,
               sin_ref, cos_ref, q_ref, k_ref, v_ref)


def _combine_qkv(y, gk, res, g, b, wq_p, wk_p, wv, bq_p, bk_p, bv,
                 sin_f, cos_f):
  hshape = jax.ShapeDtypeStruct((B, H, S, HD), jnp.float32)
  vshape = jax.ShapeDtypeStruct((B, H, S, 2 * HD), jnp.float32)
  mspec = lambda n: pl.BlockSpec((MBLK, n), lambda i: (i, 0))
  gkspec = pl.BlockSpec((MBLK, 1), lambda i: (i, 0))
  wspec = pl.BlockSpec((D, D), lambda i: (0, 0))
  bspec = pl.BlockSpec((1, D), lambda i: (0, 0))
  NQB = S // MBLK
  ospec = pl.BlockSpec((1, H, MBLK, HD), lambda i: (i // NQB, 0, i % NQB, 0))
  vspec = pl.BlockSpec((1, H, MBLK, 2 * HD),
                       lambda i: (i // NQB, 0, i % NQB, 0))
  return pl.pallas_call(
      _combine_qkv_body,
      grid=(NMB,),
      in_specs=[mspec(D), gkspec, mspec(D), bspec, bspec,
                wspec, wspec, wspec, bspec, bspec, bspec,
                mspec(D // 2), mspec(D // 2)],
      out_specs=[mspec(D), ospec, ospec, vspec],
      out_shape=[jax.ShapeDtypeStruct((T, D), jnp.float32),
                 hshape, hshape, vshape],
      compiler_params=pltpu.CompilerParams(
          dimension_semantics=("arbitrary",)),
  )(y, gk, res, g, b, wq_p, wk_p, wv, bq_p, bk_p, bv, sin_f, cos_f)


def _combine_final_loss_body(y_ref, gk_ref, res_ref, g_ref, b_ref,
                             gf_ref, bf_ref, l1_ref, l2_ref, e1_ref, e2_ref,
                             o_ref, z_ref, aux_ref, acc_ref):
  i = pl.program_id(0)

  @pl.when(i == 0)
  def _():
    acc_ref[...] = jnp.zeros((2, E), jnp.float32)

  t = res_ref[...] + y_ref[...] * gk_ref[...]
  t = _ln(t, g_ref[...], b_ref[...])
  o_ref[...] = _ln(t, gf_ref[...], bf_ref[...])

  zsum = jnp.float32(0.0)
  psum = jnp.zeros((1, E), jnp.float32)
  msum = jnp.zeros((1, E), jnp.float32)
  for l_ref, e_ref in ((l1_ref, e1_ref), (l2_ref, e2_ref)):
    logits = l_ref[...]
    m = jnp.max(logits, axis=-1, keepdims=True)
    ex = jnp.exp(logits - m)
    se = jnp.sum(ex, axis=-1, keepdims=True)
    lse = m + jnp.log(se)
    zsum = zsum + jnp.sum(lse * lse)
    psum = psum + jnp.sum(ex / se, axis=0, keepdims=True)
    iota_e = lax.broadcasted_iota(jnp.int32, (MBLK, E), 1)
    msum = msum + jnp.sum((iota_e == e_ref[...]).astype(jnp.float32),
                          axis=0, keepdims=True)
  acc_ref[0:1, :] = acc_ref[0:1, :] + psum
  acc_ref[1:2, :] = acc_ref[1:2, :] + msum
  zprev = jnp.where(i == 0, 0.0, z_ref[0, 0])
  z_ref[...] = (zprev + zsum).reshape(1, 1)

  @pl.when(i == NMB - 1)
  def _():
    n = jnp.float32(L * T)
    z_ref[...] = (z_ref[0, 0] / n).reshape(1, 1)
    pe = acc_ref[0:1, :] / n
    me = acc_ref[1:2, :] / n
    aux_ref[...] = (jnp.sum(me * pe) * ((E * E) / E)).reshape(1, 1)


def _combine_final_loss(y, gk, res, g, b, gf, bf, l1, l2, e1, e2):
  mspec = pl.BlockSpec((MBLK, D), lambda i: (i, 0))
  gkspec = pl.BlockSpec((MBLK, 1), lambda i: (i, 0))
  espec = pl.BlockSpec((MBLK, E), lambda i: (i, 0))
  bspec = pl.BlockSpec((1, D), lambda i: (0, 0))
  sspec = pl.BlockSpec((1, 1), lambda i: (0, 0))
  return pl.pallas_call(
      _combine_final_loss_body,
      grid=(NMB,),
      in_specs=[mspec, gkspec, mspec, bspec, bspec, bspec, bspec,
                espec, espec, gkspec, gkspec],
      out_specs=[mspec, sspec, sspec],
      out_shape=[jax.ShapeDtypeStruct((T, D), jnp.float32),
                 jax.ShapeDtypeStruct((1, 1), jnp.float32),
                 jax.ShapeDtypeStruct((1, 1), jnp.float32)],
      scratch_shapes=[pltpu.VMEM((2, E), jnp.float32)],
      compiler_params=pltpu.CompilerParams(
          dimension_semantics=("arbitrary",)),
  )(y, gk, res, g, b, gf, bf, l1, l2, e1, e2)


def _rope_tables():
  inv = 1.0 / (10000.0 ** (np.arange(0, HD, 2, dtype=np.float64) / HD))
  ang = np.arange(S, dtype=np.float64)[:, None] * inv[None, :]
  sin = np.asarray(np.sin(ang), np.float32)
  cos = np.asarray(np.cos(ang), np.float32)
  sin_f = np.tile(np.tile(sin, (1, H)), (B, 1))
  cos_f = np.tile(np.tile(cos, (1, H)), (B, 1))
  return jnp.asarray(sin_f), jnp.asarray(cos_f)


_EVEN_ODD_PERM = np.concatenate([
    np.arange(D).reshape(H, HD)[:, 0::2].reshape(-1),
    np.arange(D).reshape(H, HD)[:, 1::2].reshape(-1),
])


def kernel(input_ids, params):
  p = params
  ids = input_ids.reshape(T).astype(jnp.int32)
  sin_f, cos_f = _rope_tables()

  x = _sc_gather(p['embed'], ids)

  def layer_w(l):
    wq_p = p['Wq'][l][:, _EVEN_ODD_PERM]
    wk_p = p['Wk'][l][:, _EVEN_ODD_PERM]
    bq_p = p['bq'][l][_EVEN_ODD_PERM].reshape(1, D)
    bk_p = p['bk'][l][_EVEN_ODD_PERM].reshape(1, D)
    return (wq_p, wk_p, p['Wv'][l], bq_p, bk_p, p['bv'][l].reshape(1, D),
            sin_f, cos_f)

  q, k, v = _qkv(x, *layer_w(0))
  a = _attention(q, k, v)
  x1, logits0, eidx0, gatek0, destd0, destc0 = _proj_router(
      a, p['Wo'][0], p['bo'][0].reshape(1, D), x,
      p['ln1_g'][0].reshape(1, D), p['ln1_b'][0].reshape(1, D), p['Wr'][0])
  xe = _sc_scatter(x1, destd0.reshape(T), T + _SC_NW)
  ye = _ffn(xe[:T].reshape(E, CAP, D), p['W1'][0],
            p['b1'][0].reshape(E, 1, F), p['W2'][0],
            p['b2'][0].reshape(E, 1, D))
  y = _sc_gather(ye.reshape(E * CAP, D), destc0.reshape(T))

  x2, q, k, v = _combine_qkv(y, gatek0, x1, p['ln2_g'][0].reshape(1, D),
                             p['ln2_b'][0].reshape(1, D), *layer_w(1))
  a = _attention(q, k, v)
  x3, logits1, eidx1, gatek1, destd1, destc1 = _proj_router(
      a, p['Wo'][1], p['bo'][1].reshape(1, D), x2,
      p['ln1_g'][1].reshape(1, D), p['ln1_b'][1].reshape(1, D), p['Wr'][1])
  xe = _sc_scatter(x3, destd1.reshape(T), T + _SC_NW)
  ye = _ffn(xe[:T].reshape(E, CAP, D), p['W1'][1],
            p['b1'][1].reshape(E, 1, F), p['W2'][1],
            p['b2'][1].reshape(E, 1, D))
  y = _sc_gather(ye.reshape(E * CAP, D), destc1.reshape(T))

  xout, z, aux = _combine_final_loss(
      y, gatek1, x3, p['ln2_g'][1].reshape(1, D), p['ln2_b'][1].reshape(1, D),
      p['final_g'].reshape(1, D), p['final_b'].reshape(1, D),
      logits0, logits1, eidx0, eidx1)
  return xout.reshape(B, S, D), z[0, 0], aux[0, 0]


# speedup vs baseline: 2.3353x; 1.0079x over previous
import functools
import numpy as np
import jax
import jax.numpy as jnp
from jax import lax
from jax.experimental import pallas as pl
from jax.experimental.pallas import tpu as pltpu, tpu_sc as plsc

B = 2
S = 2048
D = 768
F = 3072
L = 2
H = 12
E = 8
CAP = 512
HD = D // H
HF = HD // 2
T = B * S
MBLK = 512
NMB = T // MBLK
_HPG = 4

_SC_NW = 32
_SC_BPW = T // _SC_NW


def _sc_gather(table, idx):
  V, Dd = table.shape
  mesh = plsc.VectorSubcoreMesh(core_axis_name="c", subcore_axis_name="s")

  @functools.partial(
      pl.kernel, mesh=mesh,
      out_type=jax.ShapeDtypeStruct((T, Dd), jnp.float32),
      scratch_types=[
          pltpu.VMEM((_SC_BPW,), jnp.int32),
          pltpu.VMEM((_SC_BPW, Dd), jnp.float32),
          pltpu.SemaphoreType.DMA,
      ],
  )
  def k(table_hbm, idx_hbm, out_hbm, idx_v, rows_v, sem):
    wid = lax.axis_index("s") * 2 + lax.axis_index("c")
    base = wid * _SC_BPW
    pltpu.sync_copy(idx_hbm.at[pl.ds(base, _SC_BPW)], idx_v)
    pltpu.async_copy(table_hbm.at[idx_v], rows_v, sem).wait()
    pltpu.sync_copy(rows_v, out_hbm.at[pl.ds(base, _SC_BPW)])

  return k(table, idx)


def _sc_scatter(rows, dest, nrows):
  Dd = rows.shape[1]
  mesh = plsc.VectorSubcoreMesh(core_axis_name="c", subcore_axis_name="s")

  @functools.partial(
      pl.kernel, mesh=mesh,
      out_type=jax.ShapeDtypeStruct((nrows, Dd), jnp.float32),
      scratch_types=[
          pltpu.VMEM((_SC_BPW,), jnp.int32),
          pltpu.VMEM((_SC_BPW, Dd), jnp.float32),
          pltpu.SemaphoreType.DMA,
      ],
  )
  def k(rows_hbm, dest_hbm, out_hbm, idx_v, rows_v, sem):
    wid = lax.axis_index("s") * 2 + lax.axis_index("c")
    base = wid * _SC_BPW
    pltpu.sync_copy(dest_hbm.at[pl.ds(base, _SC_BPW)], idx_v)
    pltpu.sync_copy(rows_hbm.at[pl.ds(base, _SC_BPW)], rows_v)
    pltpu.async_copy(rows_v, out_hbm.at[idx_v], sem).wait()

  return k(rows, dest)


def _qkv_compute(x, wq_ref, wk_ref, wv_ref, bq_ref, bk_ref, bv_ref,
                 sin_ref, cos_ref, q_ref, k_ref, v_ref):
  s = sin_ref[...]
  c = cos_ref[...]

  q = jnp.dot(x, wq_ref[...], preferred_element_type=jnp.float32) + bq_ref[...]
  q1 = q[:, :D // 2]
  q2 = q[:, D // 2:]
  qr1 = q1 * c - q2 * s
  qr2 = q1 * s + q2 * c

  k = jnp.dot(x, wk_ref[...], preferred_element_type=jnp.float32) + bk_ref[...]
  k1 = k[:, :D // 2]
  k2 = k[:, D // 2:]
  kr1 = k1 * c - k2 * s
  kr2 = k1 * s + k2 * c

  v = jnp.dot(x, wv_ref[...], preferred_element_type=jnp.float32) + bv_ref[...]

  scale = np.float32(1.0 / np.sqrt(HD))
  ones = jnp.ones((MBLK, 1), jnp.float32)
  for h in range(H):
    hf = slice(h * HF, (h + 1) * HF)
    q_ref[0, h, :, :HF] = qr1[:, hf] * scale
    q_ref[0, h, :, HF:] = qr2[:, hf] * scale
    k_ref[0, h, :, :HF] = kr1[:, hf]
    k_ref[0, h, :, HF:] = kr2[:, hf]
    v_ref[0, h, :, :HD] = v[:, h * HD:(h + 1) * HD]
    v_ref[0, h, :, HD:HD + 1] = ones


def _qkv_body(x_ref, wq_ref, wk_ref, wv_ref, bq_ref, bk_ref, bv_ref,
              sin_ref, cos_ref, q_ref, k_ref, v_ref):
  _qkv_compute(x_ref[...], wq_ref, wk_ref, wv_ref, bq_ref, bk_ref, bv_ref,
               sin_ref, cos_ref, q_ref, k_ref, v_ref)


def _qkv(x, wq_p, wk_p, wv, bq_p, bk_p, bv, sin_f, cos_f):
  hshape = jax.ShapeDtypeStruct((B, H, S, HD), jnp.float32)
  vshape = jax.ShapeDtypeStruct((B, H, S, 2 * HD), jnp.float32)
  mspec = lambda n: pl.BlockSpec((MBLK, n), lambda i: (i, 0))
  wspec = pl.BlockSpec((D, D), lambda i: (0, 0))
  bspec = pl.BlockSpec((1, D), lambda i: (0, 0))
  NQB = S // MBLK
  ospec = pl.BlockSpec((1, H, MBLK, HD), lambda i: (i // NQB, 0, i % NQB, 0))
  vspec = pl.BlockSpec((1, H, MBLK, 2 * HD),
                       lambda i: (i // NQB, 0, i % NQB, 0))
  return pl.pallas_call(
      _qkv_body,
      grid=(NMB,),
      in_specs=[mspec(D), wspec, wspec, wspec, bspec, bspec, bspec,
                mspec(D // 2), mspec(D // 2)],
      out_specs=[ospec, ospec, vspec],
      out_shape=[hshape, hshape, vshape],
      compiler_params=pltpu.CompilerParams(
          dimension_semantics=("arbitrary",)),
  )(x, wq_p, wk_p, wv, bq_p, bk_p, bv, sin_f, cos_f)


def _attn_body(q_ref, k_ref, v_ref, o_ref):
  dn = (((1,), (1,)), ((), ()))
  for h in range(_HPG):
    s = lax.dot_general(q_ref[0, h], k_ref[0, h], dn,
                        preferred_element_type=jnp.float32)
    p = jnp.exp(s)
    ol = jnp.dot(p, v_ref[0, h], preferred_element_type=jnp.float32)
    o_ref[0, h] = ol[:, :HD] / ol[:, HD:HD + 1]


def _attention(q, k, v):
  QBLK = 1024
  nq = S // QBLK
  ng = H // _HPG
  qspec = pl.BlockSpec((1, _HPG, QBLK, HD), lambda b, g, i: (b, g, i, 0))
  kspec = pl.BlockSpec((1, _HPG, S, HD), lambda b, g, i: (b, g, 0, 0))
  vspec = pl.BlockSpec((1, _HPG, S, 2 * HD), lambda b, g, i: (b, g, 0, 0))
  ospec = pl.BlockSpec((1, _HPG, QBLK, HD), lambda b, g, i: (b, g, i, 0))
  return pl.pallas_call(
      _attn_body,
      grid=(B, ng, nq),
      in_specs=[qspec, kspec, vspec],
      out_specs=ospec,
      out_shape=jax.ShapeDtypeStruct((B, H, S, HD), jnp.float32),
      compiler_params=pltpu.CompilerParams(
          dimension_semantics=("arbitrary", "arbitrary", "arbitrary")),
  )(q, k, v)


def _ln(t, g, b):
  m = jnp.mean(t, axis=-1, keepdims=True)
  d = t - m
  var = jnp.mean(d * d, axis=-1, keepdims=True)
  return d * lax.rsqrt(var + 1e-5) * g + b


def _proj_router_body(a_ref, wo_ref, bo_ref, res_ref, g_ref, b_ref, wr_ref,
                      x1_ref, logits_ref, eidx_ref, gatek_ref,
                      destd_ref, destc_ref, carry_ref):
  i = pl.program_id(0)

  @pl.when(i == 0)
  def _():
    carry_ref[...] = jnp.zeros((1, E), jnp.float32)

  a = jnp.concatenate([a_ref[0, h] for h in range(H)], axis=-1)
  t = jnp.dot(a, wo_ref[...], preferred_element_type=jnp.float32)
  t = t + bo_ref[...] + res_ref[...]
  x1 = _ln(t, g_ref[...], b_ref[...])
  x1_ref[...] = x1

  logits = jnp.dot(x1, wr_ref[...], preferred_element_type=jnp.float32)
  logits_ref[...] = logits
  m = jnp.max(logits, axis=-1, keepdims=True)
  p = jnp.exp(logits - m)
  p = p / jnp.sum(p, axis=-1, keepdims=True)
  gate = jnp.max(p, axis=-1, keepdims=True)
  iota_e = lax.broadcasted_iota(jnp.int32, (MBLK, E), 1)
  eidx = jnp.min(jnp.where(p == gate, iota_e, E), axis=-1,
                 keepdims=True)
  eidx_ref[...] = eidx
  oh = (iota_e == eidx).astype(jnp.float32)

  r = lax.broadcasted_iota(jnp.int32, (MBLK, MBLK), 0)
  cc = lax.broadcasted_iota(jnp.int32, (MBLK, MBLK), 1)
  tril = (r >= cc).astype(jnp.float32)
  cum = jnp.dot(tril, oh, preferred_element_type=jnp.float32) + carry_ref[...]
  carry_ref[...] = carry_ref[...] + jnp.sum(oh, axis=0, keepdims=True)

  pos = (jnp.sum(cum * oh, axis=-1, keepdims=True) - 1.0).astype(jnp.int32)
  keep = pos < CAP
  gatek_ref[...] = gate * keep.astype(jnp.float32)
  slot = eidx * CAP + pos
  tok = i * MBLK + lax.broadcasted_iota(jnp.int32, (MBLK, 1), 0)
  trash = T + tok // _SC_BPW
  destd_ref[...] = jnp.where(keep, slot, trash)
  destc_ref[...] = eidx * CAP + jnp.minimum(pos, CAP - 1)


def _proj_router(a, wo, bo, res, g, b, wr):
  NQB = S // MBLK
  aspec = pl.BlockSpec((1, H, MBLK, HD), lambda i: (i // NQB, 0, i % NQB, 0))
  mspec = pl.BlockSpec((MBLK, D), lambda i: (i, 0))
  espec = pl.BlockSpec((MBLK, E), lambda i: (i, 0))
  sspec = pl.BlockSpec((MBLK, 1), lambda i: (i, 0))
  wspec = pl.BlockSpec((D, D), lambda i: (0, 0))
  bspec = pl.BlockSpec((1, D), lambda i: (0, 0))
  rspec = pl.BlockSpec((D, E), lambda i: (0, 0))
  full = lambda shp, dt: jax.ShapeDtypeStruct(shp, dt)
  return pl.pallas_call(
      _proj_router_body,
      grid=(NMB,),
      in_specs=[aspec, wspec, bspec, mspec, bspec, bspec, rspec],
      out_specs=[mspec, espec, sspec, sspec, sspec, sspec],
      out_shape=[full((T, D), jnp.float32), full((T, E), jnp.float32),
                 full((T, 1), jnp.int32), full((T, 1), jnp.float32),
                 full((T, 1), jnp.int32), full((T, 1), jnp.int32)],
      scratch_shapes=[pltpu.VMEM((1, E), jnp.float32)],
      compiler_params=pltpu.CompilerParams(
          dimension_semantics=("arbitrary",)),
  )(a, wo, bo, res, g, b, wr)


def _gelu(x):
  return x * 0.5 * (1.0 + lax.erf(x * np.float32(1.0 / np.sqrt(2.0))))


def _ffn_body(xe_ref, w1_ref, b1_ref, w2_ref, b2_ref, ye_ref):
  f = pl.program_id(1)
  h = _gelu(jnp.dot(xe_ref[0].astype(w1_ref.dtype), w1_ref[0],
                    preferred_element_type=jnp.float32) + b1_ref[0])
  acc = jnp.dot(h.astype(w2_ref.dtype), w2_ref[0],
                preferred_element_type=jnp.float32)

  @pl.when(f == 0)
  def _():
    ye_ref[0] = acc + b2_ref[0]

  @pl.when(f != 0)
  def _():
    ye_ref[0] = ye_ref[0] + acc


def _ffn(xe, w1, b1, w2, b2):
  FBLK = 768
  nf = F // FBLK
  return pl.pallas_call(
      _ffn_body,
      grid=(E, nf),
      in_specs=[
          pl.BlockSpec((1, CAP, D), lambda e, f: (e, 0, 0)),
          pl.BlockSpec((1, D, FBLK), lambda e, f: (e, 0, f)),
          pl.BlockSpec((1, 1, FBLK), lambda e, f: (e, 0, f)),
          pl.BlockSpec((1, FBLK, D), lambda e, f: (e, f, 0)),
          pl.BlockSpec((1, 1, D), lambda e, f: (e, 0, 0)),
      ],
      out_specs=pl.BlockSpec((1, CAP, D), lambda e, f: (e, 0, 0)),
      out_shape=jax.ShapeDtypeStruct((E, CAP, D), jnp.float32),
      compiler_params=pltpu.CompilerParams(
          dimension_semantics=("arbitrary", "arbitrary")),
  )(xe, w1, b1, w2, b2)


def _combine_qkv_body(y_ref, gk_ref, res_ref, g_ref, b_ref,
                      wq_ref, wk_ref, wv_ref, bq_ref, bk_ref, bv_ref,
                      sin_ref, cos_ref, x_ref, q_ref, k_ref, v_ref):
  t = res_ref[...] + y_ref[...] * gk_ref[...]
  x = _ln(t, g_ref[...], b_ref[...])
  x_ref[...] = x
  _qkv_compute(x, wq_ref, wk_ref, wv_ref, bq_ref, bk_ref, bv_ref,
               sin_ref, cos_ref, q_ref, k_ref, v_ref)


def _combine_qkv(y, gk, res, g, b, wq_p, wk_p, wv, bq_p, bk_p, bv,
                 sin_f, cos_f):
  hshape = jax.ShapeDtypeStruct((B, H, S, HD), jnp.float32)
  vshape = jax.ShapeDtypeStruct((B, H, S, 2 * HD), jnp.float32)
  mspec = lambda n: pl.BlockSpec((MBLK, n), lambda i: (i, 0))
  gkspec = pl.BlockSpec((MBLK, 1), lambda i: (i, 0))
  wspec = pl.BlockSpec((D, D), lambda i: (0, 0))
  bspec = pl.BlockSpec((1, D), lambda i: (0, 0))
  NQB = S // MBLK
  ospec = pl.BlockSpec((1, H, MBLK, HD), lambda i: (i // NQB, 0, i % NQB, 0))
  vspec = pl.BlockSpec((1, H, MBLK, 2 * HD),
                       lambda i: (i // NQB, 0, i % NQB, 0))
  return pl.pallas_call(
      _combine_qkv_body,
      grid=(NMB,),
      in_specs=[mspec(D), gkspec, mspec(D), bspec, bspec,
                wspec, wspec, wspec, bspec, bspec, bspec,
                mspec(D // 2), mspec(D // 2)],
      out_specs=[mspec(D), ospec, ospec, vspec],
      out_shape=[jax.ShapeDtypeStruct((T, D), jnp.float32),
                 hshape, hshape, vshape],
      compiler_params=pltpu.CompilerParams(
          dimension_semantics=("arbitrary",)),
  )(y, gk, res, g, b, wq_p, wk_p, wv, bq_p, bk_p, bv, sin_f, cos_f)


def _combine_final_loss_body(y_ref, gk_ref, res_ref, g_ref, b_ref,
                             gf_ref, bf_ref, l1_ref, l2_ref, e1_ref, e2_ref,
                             o_ref, z_ref, aux_ref, acc_ref):
  i = pl.program_id(0)

  @pl.when(i == 0)
  def _():
    acc_ref[...] = jnp.zeros((2, E), jnp.float32)

  t = res_ref[...] + y_ref[...] * gk_ref[...]
  t = _ln(t, g_ref[...], b_ref[...])
  o_ref[...] = _ln(t, gf_ref[...], bf_ref[...])

  zsum = jnp.float32(0.0)
  psum = jnp.zeros((1, E), jnp.float32)
  msum = jnp.zeros((1, E), jnp.float32)
  for l_ref, e_ref in ((l1_ref, e1_ref), (l2_ref, e2_ref)):
    logits = l_ref[...]
    m = jnp.max(logits, axis=-1, keepdims=True)
    ex = jnp.exp(logits - m)
    se = jnp.sum(ex, axis=-1, keepdims=True)
    lse = m + jnp.log(se)
    zsum = zsum + jnp.sum(lse * lse)
    psum = psum + jnp.sum(ex / se, axis=0, keepdims=True)
    iota_e = lax.broadcasted_iota(jnp.int32, (MBLK, E), 1)
    msum = msum + jnp.sum((iota_e == e_ref[...]).astype(jnp.float32),
                          axis=0, keepdims=True)
  acc_ref[0:1, :] = acc_ref[0:1, :] + psum
  acc_ref[1:2, :] = acc_ref[1:2, :] + msum
  zprev = jnp.where(i == 0, 0.0, z_ref[0, 0])
  z_ref[...] = (zprev + zsum).reshape(1, 1)

  @pl.when(i == NMB - 1)
  def _():
    n = jnp.float32(L * T)
    z_ref[...] = (z_ref[0, 0] / n).reshape(1, 1)
    pe = acc_ref[0:1, :] / n
    me = acc_ref[1:2, :] / n
    aux_ref[...] = (jnp.sum(me * pe) * ((E * E) / E)).reshape(1, 1)


def _combine_final_loss(y, gk, res, g, b, gf, bf, l1, l2, e1, e2):
  mspec = pl.BlockSpec((MBLK, D), lambda i: (i, 0))
  gkspec = pl.BlockSpec((MBLK, 1), lambda i: (i, 0))
  espec = pl.BlockSpec((MBLK, E), lambda i: (i, 0))
  bspec = pl.BlockSpec((1, D), lambda i: (0, 0))
  sspec = pl.BlockSpec((1, 1), lambda i: (0, 0))
  return pl.pallas_call(
      _combine_final_loss_body,
      grid=(NMB,),
      in_specs=[mspec, gkspec, mspec, bspec, bspec, bspec, bspec,
                espec, espec, gkspec, gkspec],
      out_specs=[mspec, sspec, sspec],
      out_shape=[jax.ShapeDtypeStruct((T, D), jnp.float32),
                 jax.ShapeDtypeStruct((1, 1), jnp.float32),
                 jax.ShapeDtypeStruct((1, 1), jnp.float32)],
      scratch_shapes=[pltpu.VMEM((2, E), jnp.float32)],
      compiler_params=pltpu.CompilerParams(
          dimension_semantics=("arbitrary",)),
  )(y, gk, res, g, b, gf, bf, l1, l2, e1, e2)


def _rope_tables():
  inv = 1.0 / (10000.0 ** (np.arange(0, HD, 2, dtype=np.float64) / HD))
  ang = np.arange(S, dtype=np.float64)[:, None] * inv[None, :]
  sin = np.asarray(np.sin(ang), np.float32)
  cos = np.asarray(np.cos(ang), np.float32)
  sin_f = np.tile(np.tile(sin, (1, H)), (B, 1))
  cos_f = np.tile(np.tile(cos, (1, H)), (B, 1))
  return jnp.asarray(sin_f), jnp.asarray(cos_f)


_EVEN_ODD_PERM = np.concatenate([
    np.arange(D).reshape(H, HD)[:, 0::2].reshape(-1),
    np.arange(D).reshape(H, HD)[:, 1::2].reshape(-1),
])


def kernel(input_ids, params):
  p = params
  ids = input_ids.reshape(T).astype(jnp.int32)
  sin_f, cos_f = _rope_tables()

  x = _sc_gather(p['embed'], ids)

  def layer_w(l):
    wq_p = p['Wq'][l][:, _EVEN_ODD_PERM]
    wk_p = p['Wk'][l][:, _EVEN_ODD_PERM]
    bq_p = p['bq'][l][_EVEN_ODD_PERM].reshape(1, D)
    bk_p = p['bk'][l][_EVEN_ODD_PERM].reshape(1, D)
    return (wq_p, wk_p, p['Wv'][l], bq_p, bk_p, p['bv'][l].reshape(1, D),
            sin_f, cos_f)

  q, k, v = _qkv(x, *layer_w(0))
  a = _attention(q, k, v)
  x1, logits0, eidx0, gatek0, destd0, destc0 = _proj_router(
      a, p['Wo'][0], p['bo'][0].reshape(1, D), x,
      p['ln1_g'][0].reshape(1, D), p['ln1_b'][0].reshape(1, D), p['Wr'][0])
  xe = _sc_scatter(x1, destd0.reshape(T), T + _SC_NW)
  ye = _ffn(xe[:T].reshape(E, CAP, D), p['W1'][0],
            p['b1'][0].reshape(E, 1, F), p['W2'][0],
            p['b2'][0].reshape(E, 1, D))
  y = _sc_gather(ye.reshape(E * CAP, D), destc0.reshape(T))

  x2, q, k, v = _combine_qkv(y, gatek0, x1, p['ln2_g'][0].reshape(1, D),
                             p['ln2_b'][0].reshape(1, D), *layer_w(1))
  a = _attention(q, k, v)
  x3, logits1, eidx1, gatek1, destd1, destc1 = _proj_router(
      a, p['Wo'][1], p['bo'][1].reshape(1, D), x2,
      p['ln1_g'][1].reshape(1, D), p['ln1_b'][1].reshape(1, D), p['Wr'][1])
  xe = _sc_scatter(x3, destd1.reshape(T), T + _SC_NW)
  ye = _ffn(xe[:T].reshape(E, CAP, D), p['W1'][1],
            p['b1'][1].reshape(E, 1, F), p['W2'][1],
            p['b2'][1].reshape(E, 1, D))
  y = _sc_gather(ye.reshape(E * CAP, D), destc1.reshape(T))

  xout, z, aux = _combine_final_loss(
      y, gatek1, x3, p['ln2_g'][1].reshape(1, D), p['ln2_b'][1].reshape(1, D),
      p['final_g'].reshape(1, D), p['final_b'].reshape(1, D),
      logits0, logits1, eidx0, eidx1)
  return xout.reshape(B, S, D), z[0, 0], aux[0, 0]


# speedup vs baseline: 2.3790x; 1.0187x over previous
import functools
import numpy as np
import jax
import jax.numpy as jnp
from jax import lax
from jax.experimental import pallas as pl
from jax.experimental.pallas import tpu as pltpu, tpu_sc as plsc

B = 2
S = 2048
D = 768
F = 3072
L = 2
H = 12
E = 8
CAP = 512
HD = D // H
HF = HD // 2
T = B * S
MBLK = 512
NMB = T // MBLK
_HPG = 4

_SC_NW = 32
_SC_BPW = T // _SC_NW


def _sc_gather(table, idx):
  V, Dd = table.shape
  mesh = plsc.VectorSubcoreMesh(core_axis_name="c", subcore_axis_name="s")

  @functools.partial(
      pl.kernel, mesh=mesh,
      out_type=jax.ShapeDtypeStruct((T, Dd), jnp.float32),
      scratch_types=[
          pltpu.VMEM((_SC_BPW,), jnp.int32),
          pltpu.VMEM((_SC_BPW, Dd), jnp.float32),
          pltpu.SemaphoreType.DMA,
      ],
  )
  def k(table_hbm, idx_hbm, out_hbm, idx_v, rows_v, sem):
    wid = lax.axis_index("s") * 2 + lax.axis_index("c")
    base = wid * _SC_BPW
    pltpu.sync_copy(idx_hbm.at[pl.ds(base, _SC_BPW)], idx_v)
    pltpu.async_copy(table_hbm.at[idx_v], rows_v, sem).wait()
    pltpu.sync_copy(rows_v, out_hbm.at[pl.ds(base, _SC_BPW)])

  return k(table, idx)


def _sc_scatter(rows, dest, nrows):
  Dd = rows.shape[1]
  mesh = plsc.VectorSubcoreMesh(core_axis_name="c", subcore_axis_name="s")

  @functools.partial(
      pl.kernel, mesh=mesh,
      out_type=jax.ShapeDtypeStruct((nrows, Dd), jnp.float32),
      scratch_types=[
          pltpu.VMEM((_SC_BPW,), jnp.int32),
          pltpu.VMEM((_SC_BPW, Dd), jnp.float32),
          pltpu.SemaphoreType.DMA,
      ],
  )
  def k(rows_hbm, dest_hbm, out_hbm, idx_v, rows_v, sem):
    wid = lax.axis_index("s") * 2 + lax.axis_index("c")
    base = wid * _SC_BPW
    pltpu.sync_copy(dest_hbm.at[pl.ds(base, _SC_BPW)], idx_v)
    pltpu.sync_copy(rows_hbm.at[pl.ds(base, _SC_BPW)], rows_v)
    pltpu.async_copy(rows_v, out_hbm.at[idx_v], sem).wait()

  return k(rows, dest)


def _qkv_compute(x, wq_ref, wk_ref, wv_ref, bq_ref, bk_ref, bv_ref,
                 sin_ref, cos_ref, q_ref, k_ref, v_ref):
  s = sin_ref[...]
  c = cos_ref[...]

  q = jnp.dot(x, wq_ref[...], preferred_element_type=jnp.float32) + bq_ref[...]
  q1 = q[:, :D // 2]
  q2 = q[:, D // 2:]
  qr1 = q1 * c - q2 * s
  qr2 = q1 * s + q2 * c

  k = jnp.dot(x, wk_ref[...], preferred_element_type=jnp.float32) + bk_ref[...]
  k1 = k[:, :D // 2]
  k2 = k[:, D // 2:]
  kr1 = k1 * c - k2 * s
  kr2 = k1 * s + k2 * c

  v = jnp.dot(x, wv_ref[...], preferred_element_type=jnp.float32) + bv_ref[...]

  scale = np.float32(1.0 / np.sqrt(HD))
  ones = jnp.ones((MBLK, 1), jnp.float32)
  for h in range(H):
    hf = slice(h * HF, (h + 1) * HF)
    q_ref[0, h, :, :HF] = qr1[:, hf] * scale
    q_ref[0, h, :, HF:] = qr2[:, hf] * scale
    k_ref[0, h, :, :HF] = kr1[:, hf]
    k_ref[0, h, :, HF:] = kr2[:, hf]
    v_ref[0, h, :, :HD] = v[:, h * HD:(h + 1) * HD]
    v_ref[0, h, :, HD:HD + 1] = ones


def _qkv_body(x_ref, wq_ref, wk_ref, wv_ref, bq_ref, bk_ref, bv_ref,
              sin_ref, cos_ref, q_ref, k_ref, v_ref):
  _qkv_compute(x_ref[...], wq_ref, wk_ref, wv_ref, bq_ref, bk_ref, bv_ref,
               sin_ref, cos_ref, q_ref, k_ref, v_ref)


def _qkv(x, wq_p, wk_p, wv, bq_p, bk_p, bv, sin_f, cos_f):
  hshape = jax.ShapeDtypeStruct((B, H, S, HD), jnp.float32)
  vshape = jax.ShapeDtypeStruct((B, H, S, 2 * HD), jnp.float32)
  mspec = lambda n: pl.BlockSpec((MBLK, n), lambda i: (i, 0))
  wspec = pl.BlockSpec((D, D), lambda i: (0, 0))
  bspec = pl.BlockSpec((1, D), lambda i: (0, 0))
  NQB = S // MBLK
  ospec = pl.BlockSpec((1, H, MBLK, HD), lambda i: (i // NQB, 0, i % NQB, 0))
  vspec = pl.BlockSpec((1, H, MBLK, 2 * HD),
                       lambda i: (i // NQB, 0, i % NQB, 0))
  return pl.pallas_call(
      _qkv_body,
      grid=(NMB,),
      in_specs=[mspec(D), wspec, wspec, wspec, bspec, bspec, bspec,
                mspec(D // 2), mspec(D // 2)],
      out_specs=[ospec, ospec, vspec],
      out_shape=[hshape, hshape, vshape],
      compiler_params=pltpu.CompilerParams(
          dimension_semantics=("arbitrary",)),
  )(x, wq_p, wk_p, wv, bq_p, bk_p, bv, sin_f, cos_f)


def _attn_body(q_ref, k_ref, v_ref, o_ref):
  dn = (((1,), (1,)), ((), ()))
  for h in range(_HPG):
    s = lax.dot_general(q_ref[0, h], k_ref[0, h], dn,
                        preferred_element_type=jnp.float32)
    p = jnp.exp(s)
    ol = jnp.dot(p, v_ref[0, h], preferred_element_type=jnp.float32)
    o_ref[0, h] = ol[:, :HD] / ol[:, HD:HD + 1]


def _attention(q, k, v):
  QBLK = 1024
  nq = S // QBLK
  ng = H // _HPG
  qspec = pl.BlockSpec((1, _HPG, QBLK, HD), lambda b, g, i: (b, g, i, 0))
  kspec = pl.BlockSpec((1, _HPG, S, HD), lambda b, g, i: (b, g, 0, 0))
  vspec = pl.BlockSpec((1, _HPG, S, 2 * HD), lambda b, g, i: (b, g, 0, 0))
  ospec = pl.BlockSpec((1, _HPG, QBLK, HD), lambda b, g, i: (b, g, i, 0))
  return pl.pallas_call(
      _attn_body,
      grid=(B, ng, nq),
      in_specs=[qspec, kspec, vspec],
      out_specs=ospec,
      out_shape=jax.ShapeDtypeStruct((B, H, S, HD), jnp.float32),
      compiler_params=pltpu.CompilerParams(
          dimension_semantics=("arbitrary", "arbitrary", "arbitrary")),
  )(q, k, v)


def _ln(t, g, b):
  m = jnp.mean(t, axis=-1, keepdims=True)
  d = t - m
  var = jnp.mean(d * d, axis=-1, keepdims=True)
  return d * lax.rsqrt(var + 1e-5) * g + b


def _proj_router_body(a_ref, wo_ref, bo_ref, res_ref, g_ref, b_ref, wr_ref,
                      x1_ref, logits_ref, eidx_ref, gatek_ref,
                      destd_ref, destc_ref, carry_ref):
  i = pl.program_id(0)

  @pl.when(i == 0)
  def _():
    carry_ref[...] = jnp.zeros((1, E), jnp.float32)

  a = jnp.concatenate([a_ref[0, h] for h in range(H)], axis=-1)
  t = jnp.dot(a, wo_ref[...], preferred_element_type=jnp.float32)
  t = t + bo_ref[...] + res_ref[...]
  x1 = _ln(t, g_ref[...], b_ref[...])
  x1_ref[...] = x1

  logits = jnp.dot(x1, wr_ref[...], preferred_element_type=jnp.float32)
  logits_ref[...] = logits
  m = jnp.max(logits, axis=-1, keepdims=True)
  p = jnp.exp(logits - m)
  p = p / jnp.sum(p, axis=-1, keepdims=True)
  gate = jnp.max(p, axis=-1, keepdims=True)
  iota_e = lax.broadcasted_iota(jnp.int32, (MBLK, E), 1)
  eidx = jnp.min(jnp.where(p == gate, iota_e, E), axis=-1,
                 keepdims=True)
  eidx_ref[...] = eidx
  oh = (iota_e == eidx).astype(jnp.float32)

  r = lax.broadcasted_iota(jnp.int32, (MBLK, MBLK), 0)
  cc = lax.broadcasted_iota(jnp.int32, (MBLK, MBLK), 1)
  tril = (r >= cc).astype(jnp.float32)
  cum = jnp.dot(tril, oh, preferred_element_type=jnp.float32) + carry_ref[...]
  carry_ref[...] = carry_ref[...] + jnp.sum(oh, axis=0, keepdims=True)

  pos = (jnp.sum(cum * oh, axis=-1, keepdims=True) - 1.0).astype(jnp.int32)
  keep = pos < CAP
  gatek_ref[...] = gate * keep.astype(jnp.float32)
  slot = eidx * CAP + pos
  tok = i * MBLK + lax.broadcasted_iota(jnp.int32, (MBLK, 1), 0)
  trash = T + tok // _SC_BPW
  destd_ref[...] = jnp.where(keep, slot, trash)
  destc_ref[...] = eidx * CAP + jnp.minimum(pos, CAP - 1)


def _proj_router(a, wo, bo, res, g, b, wr):
  NQB = S // MBLK
  aspec = pl.BlockSpec((1, H, MBLK, HD), lambda i: (i // NQB, 0, i % NQB, 0))
  mspec = pl.BlockSpec((MBLK, D), lambda i: (i, 0))
  espec = pl.BlockSpec((MBLK, E), lambda i: (i, 0))
  sspec = pl.BlockSpec((MBLK, 1), lambda i: (i, 0))
  wspec = pl.BlockSpec((D, D), lambda i: (0, 0))
  bspec = pl.BlockSpec((1, D), lambda i: (0, 0))
  rspec = pl.BlockSpec((D, E), lambda i: (0, 0))
  full = lambda shp, dt: jax.ShapeDtypeStruct(shp, dt)
  return pl.pallas_call(
      _proj_router_body,
      grid=(NMB,),
      in_specs=[aspec, wspec, bspec, mspec, bspec, bspec, rspec],
      out_specs=[mspec, espec, sspec, sspec, sspec, sspec],
      out_shape=[full((T, D), jnp.float32), full((T, E), jnp.float32),
                 full((T, 1), jnp.int32), full((T, 1), jnp.float32),
                 full((T, 1), jnp.int32), full((T, 1), jnp.int32)],
      scratch_shapes=[pltpu.VMEM((1, E), jnp.float32)],
      compiler_params=pltpu.CompilerParams(
          dimension_semantics=("arbitrary",)),
  )(a, wo, bo, res, g, b, wr)


def _gelu(x):
  return x * 0.5 * (1.0 + lax.erf(x * np.float32(1.0 / np.sqrt(2.0))))


def _ffn_body(xe_ref, w1_ref, b1_ref, w2_ref, b2_ref, ye_ref):
  f = pl.program_id(1)
  h = _gelu(jnp.dot(xe_ref[0].astype(w1_ref.dtype), w1_ref[0],
                    preferred_element_type=jnp.float32) + b1_ref[0])
  acc = jnp.dot(h.astype(w2_ref.dtype), w2_ref[0],
                preferred_element_type=jnp.float32)

  @pl.when(f == 0)
  def _():
    ye_ref[0] = acc + b2_ref[0]

  @pl.when(f != 0)
  def _():
    ye_ref[0] = ye_ref[0] + acc


def _ffn(xe, w1, b1, w2, b2):
  FBLK = 1536
  nf = F // FBLK
  return pl.pallas_call(
      _ffn_body,
      grid=(E, nf),
      in_specs=[
          pl.BlockSpec((1, CAP, D), lambda e, f: (e, 0, 0)),
          pl.BlockSpec((1, D, FBLK), lambda e, f: (e, 0, f)),
          pl.BlockSpec((1, 1, FBLK), lambda e, f: (e, 0, f)),
          pl.BlockSpec((1, FBLK, D), lambda e, f: (e, f, 0)),
          pl.BlockSpec((1, 1, D), lambda e, f: (e, 0, 0)),
      ],
      out_specs=pl.BlockSpec((1, CAP, D), lambda e, f: (e, 0, 0)),
      out_shape=jax.ShapeDtypeStruct((E, CAP, D), jnp.float32),
      compiler_params=pltpu.CompilerParams(
          dimension_semantics=("arbitrary", "arbitrary")),
  )(xe, w1, b1, w2, b2)


def _combine_qkv_body(y_ref, gk_ref, res_ref, g_ref, b_ref,
                      wq_ref, wk_ref, wv_ref, bq_ref, bk_ref, bv_ref,
                      sin_ref, cos_ref, x_ref, q_ref, k_ref, v_ref):
  t = res_ref[...] + y_ref[...] * gk_ref[...]
  x = _ln(t, g_ref[...], b_ref[...])
  x_ref[...] = x
  _qkv_compute(x, wq_ref, wk_ref, wv_ref, bq_ref, bk_ref, bv_ref,
               sin_ref, cos_ref, q_ref, k_ref, v_ref)


def _combine_qkv(y, gk, res, g, b, wq_p, wk_p, wv, bq_p, bk_p, bv,
                 sin_f, cos_f):
  hshape = jax.ShapeDtypeStruct((B, H, S, HD), jnp.float32)
  vshape = jax.ShapeDtypeStruct((B, H, S, 2 * HD), jnp.float32)
  mspec = lambda n: pl.BlockSpec((MBLK, n), lambda i: (i, 0))
  gkspec = pl.BlockSpec((MBLK, 1), lambda i: (i, 0))
  wspec = pl.BlockSpec((D, D), lambda i: (0, 0))
  bspec = pl.BlockSpec((1, D), lambda i: (0, 0))
  NQB = S // MBLK
  ospec = pl.BlockSpec((1, H, MBLK, HD), lambda i: (i // NQB, 0, i % NQB, 0))
  vspec = pl.BlockSpec((1, H, MBLK, 2 * HD),
                       lambda i: (i // NQB, 0, i % NQB, 0))
  return pl.pallas_call(
      _combine_qkv_body,
      grid=(NMB,),
      in_specs=[mspec(D), gkspec, mspec(D), bspec, bspec,
                wspec, wspec, wspec, bspec, bspec, bspec,
                mspec(D // 2), mspec(D // 2)],
      out_specs=[mspec(D), ospec, ospec, vspec],
      out_shape=[jax.ShapeDtypeStruct((T, D), jnp.float32),
                 hshape, hshape, vshape],
      compiler_params=pltpu.CompilerParams(
          dimension_semantics=("arbitrary",)),
  )(y, gk, res, g, b, wq_p, wk_p, wv, bq_p, bk_p, bv, sin_f, cos_f)


def _combine_final_loss_body(y_ref, gk_ref, res_ref, g_ref, b_ref,
                             gf_ref, bf_ref, l1_ref, l2_ref, e1_ref, e2_ref,
                             o_ref, z_ref, aux_ref, acc_ref):
  i = pl.program_id(0)

  @pl.when(i == 0)
  def _():
    acc_ref[...] = jnp.zeros((2, E), jnp.float32)

  t = res_ref[...] + y_ref[...] * gk_ref[...]
  t = _ln(t, g_ref[...], b_ref[...])
  o_ref[...] = _ln(t, gf_ref[...], bf_ref[...])

  zsum = jnp.float32(0.0)
  psum = jnp.zeros((1, E), jnp.float32)
  msum = jnp.zeros((1, E), jnp.float32)
  for l_ref, e_ref in ((l1_ref, e1_ref), (l2_ref, e2_ref)):
    logits = l_ref[...]
    m = jnp.max(logits, axis=-1, keepdims=True)
    ex = jnp.exp(logits - m)
    se = jnp.sum(ex, axis=-1, keepdims=True)
    lse = m + jnp.log(se)
    zsum = zsum + jnp.sum(lse * lse)
    psum = psum + jnp.sum(ex / se, axis=0, keepdims=True)
    iota_e = lax.broadcasted_iota(jnp.int32, (MBLK, E), 1)
    msum = msum + jnp.sum((iota_e == e_ref[...]).astype(jnp.float32),
                          axis=0, keepdims=True)
  acc_ref[0:1, :] = acc_ref[0:1, :] + psum
  acc_ref[1:2, :] = acc_ref[1:2, :] + msum
  zprev = jnp.where(i == 0, 0.0, z_ref[0, 0])
  z_ref[...] = (zprev + zsum).reshape(1, 1)

  @pl.when(i == NMB - 1)
  def _():
    n = jnp.float32(L * T)
    z_ref[...] = (z_ref[0, 0] / n).reshape(1, 1)
    pe = acc_ref[0:1, :] / n
    me = acc_ref[1:2, :] / n
    aux_ref[...] = (jnp.sum(me * pe) * ((E * E) / E)).reshape(1, 1)


def _combine_final_loss(y, gk, res, g, b, gf, bf, l1, l2, e1, e2):
  mspec = pl.BlockSpec((MBLK, D), lambda i: (i, 0))
  gkspec = pl.BlockSpec((MBLK, 1), lambda i: (i, 0))
  espec = pl.BlockSpec((MBLK, E), lambda i: (i, 0))
  bspec = pl.BlockSpec((1, D), lambda i: (0, 0))
  sspec = pl.BlockSpec((1, 1), lambda i: (0, 0))
  return pl.pallas_call(
      _combine_final_loss_body,
      grid=(NMB,),
      in_specs=[mspec, gkspec, mspec, bspec, bspec, bspec, bspec,
                espec, espec, gkspec, gkspec],
      out_specs=[mspec, sspec, sspec],
      out_shape=[jax.ShapeDtypeStruct((T, D), jnp.float32),
                 jax.ShapeDtypeStruct((1, 1), jnp.float32),
                 jax.ShapeDtypeStruct((1, 1), jnp.float32)],
      scratch_shapes=[pltpu.VMEM((2, E), jnp.float32)],
      compiler_params=pltpu.CompilerParams(
          dimension_semantics=("arbitrary",)),
  )(y, gk, res, g, b, gf, bf, l1, l2, e1, e2)


def _rope_tables():
  inv = 1.0 / (10000.0 ** (np.arange(0, HD, 2, dtype=np.float64) / HD))
  ang = np.arange(S, dtype=np.float64)[:, None] * inv[None, :]
  sin = np.asarray(np.sin(ang), np.float32)
  cos = np.asarray(np.cos(ang), np.float32)
  sin_f = np.tile(np.tile(sin, (1, H)), (B, 1))
  cos_f = np.tile(np.tile(cos, (1, H)), (B, 1))
  return jnp.asarray(sin_f), jnp.asarray(cos_f)


_EVEN_ODD_PERM = np.concatenate([
    np.arange(D).reshape(H, HD)[:, 0::2].reshape(-1),
    np.arange(D).reshape(H, HD)[:, 1::2].reshape(-1),
])


def kernel(input_ids, params):
  p = params
  ids = input_ids.reshape(T).astype(jnp.int32)
  sin_f, cos_f = _rope_tables()

  x = _sc_gather(p['embed'], ids)

  def layer_w(l):
    wq_p = p['Wq'][l][:, _EVEN_ODD_PERM]
    wk_p = p['Wk'][l][:, _EVEN_ODD_PERM]
    bq_p = p['bq'][l][_EVEN_ODD_PERM].reshape(1, D)
    bk_p = p['bk'][l][_EVEN_ODD_PERM].reshape(1, D)
    return (wq_p, wk_p, p['Wv'][l], bq_p, bk_p, p['bv'][l].reshape(1, D),
            sin_f, cos_f)

  q, k, v = _qkv(x, *layer_w(0))
  a = _attention(q, k, v)
  x1, logits0, eidx0, gatek0, destd0, destc0 = _proj_router(
      a, p['Wo'][0], p['bo'][0].reshape(1, D), x,
      p['ln1_g'][0].reshape(1, D), p['ln1_b'][0].reshape(1, D), p['Wr'][0])
  xe = _sc_scatter(x1, destd0.reshape(T), T + _SC_NW)
  ye = _ffn(xe[:T].reshape(E, CAP, D), p['W1'][0],
            p['b1'][0].reshape(E, 1, F), p['W2'][0],
            p['b2'][0].reshape(E, 1, D))
  y = _sc_gather(ye.reshape(E * CAP, D), destc0.reshape(T))

  x2, q, k, v = _combine_qkv(y, gatek0, x1, p['ln2_g'][0].reshape(1, D),
                             p['ln2_b'][0].reshape(1, D), *layer_w(1))
  a = _attention(q, k, v)
  x3, logits1, eidx1, gatek1, destd1, destc1 = _proj_router(
      a, p['Wo'][1], p['bo'][1].reshape(1, D), x2,
      p['ln1_g'][1].reshape(1, D), p['ln1_b'][1].reshape(1, D), p['Wr'][1])
  xe = _sc_scatter(x3, destd1.reshape(T), T + _SC_NW)
  ye = _ffn(xe[:T].reshape(E, CAP, D), p['W1'][1],
            p['b1'][1].reshape(E, 1, F), p['W2'][1],
            p['b2'][1].reshape(E, 1, D))
  y = _sc_gather(ye.reshape(E * CAP, D), destc1.reshape(T))

  xout, z, aux = _combine_final_loss(
      y, gatek1, x3, p['ln2_g'][1].reshape(1, D), p['ln2_b'][1].reshape(1, D),
      p['final_g'].reshape(1, D), p['final_b'].reshape(1, D),
      logits0, logits1, eidx0, eidx1)
  return xout.reshape(B, S, D), z[0, 0], aux[0, 0]


# speedup vs baseline: 2.4648x; 1.0361x over previous
import functools
import numpy as np
import jax
import jax.numpy as jnp
from jax import lax
from jax.experimental import pallas as pl
from jax.experimental.pallas import tpu as pltpu, tpu_sc as plsc

B = 2
S = 2048
D = 768
F = 3072
L = 2
H = 12
E = 8
CAP = 512
HD = D // H
HF = HD // 2
T = B * S
MBLK = 512
NMB = T // MBLK
_HPG = 4

_SC_NW = 32
_SC_BPW = T // _SC_NW


def _sc_gather(table, idx):
  V, Dd = table.shape
  mesh = plsc.VectorSubcoreMesh(core_axis_name="c", subcore_axis_name="s")

  @functools.partial(
      pl.kernel, mesh=mesh,
      out_type=jax.ShapeDtypeStruct((T, Dd), jnp.float32),
      scratch_types=[
          pltpu.VMEM((_SC_BPW,), jnp.int32),
          pltpu.VMEM((_SC_BPW, Dd), jnp.float32),
          pltpu.SemaphoreType.DMA,
      ],
  )
  def k(table_hbm, idx_hbm, out_hbm, idx_v, rows_v, sem):
    wid = lax.axis_index("s") * 2 + lax.axis_index("c")
    base = wid * _SC_BPW
    pltpu.sync_copy(idx_hbm.at[pl.ds(base, _SC_BPW)], idx_v)
    pltpu.async_copy(table_hbm.at[idx_v], rows_v, sem).wait()
    pltpu.sync_copy(rows_v, out_hbm.at[pl.ds(base, _SC_BPW)])

  return k(table, idx)


def _sc_scatter(rows, dest, nrows):
  Dd = rows.shape[1]
  mesh = plsc.VectorSubcoreMesh(core_axis_name="c", subcore_axis_name="s")

  @functools.partial(
      pl.kernel, mesh=mesh,
      out_type=jax.ShapeDtypeStruct((nrows, Dd), jnp.float32),
      scratch_types=[
          pltpu.VMEM((_SC_BPW,), jnp.int32),
          pltpu.VMEM((_SC_BPW, Dd), jnp.float32),
          pltpu.SemaphoreType.DMA,
      ],
  )
  def k(rows_hbm, dest_hbm, out_hbm, idx_v, rows_v, sem):
    wid = lax.axis_index("s") * 2 + lax.axis_index("c")
    base = wid * _SC_BPW
    pltpu.sync_copy(dest_hbm.at[pl.ds(base, _SC_BPW)], idx_v)
    pltpu.sync_copy(rows_hbm.at[pl.ds(base, _SC_BPW)], rows_v)
    pltpu.async_copy(rows_v, out_hbm.at[idx_v], sem).wait()

  return k(rows, dest)


def _qkv_compute(x, wq_ref, wk_ref, wv_ref, bq_ref, bk_ref, bv_ref,
                 sin_ref, cos_ref, q_ref, k_ref, v_ref):
  s = sin_ref[...]
  c = cos_ref[...]

  q = jnp.dot(x, wq_ref[...], preferred_element_type=jnp.float32) + bq_ref[...]
  q1 = q[:, :D // 2]
  q2 = q[:, D // 2:]
  qr1 = q1 * c - q2 * s
  qr2 = q1 * s + q2 * c

  k = jnp.dot(x, wk_ref[...], preferred_element_type=jnp.float32) + bk_ref[...]
  k1 = k[:, :D // 2]
  k2 = k[:, D // 2:]
  kr1 = k1 * c - k2 * s
  kr2 = k1 * s + k2 * c

  v = jnp.dot(x, wv_ref[...], preferred_element_type=jnp.float32) + bv_ref[...]

  scale = np.float32(1.0 / np.sqrt(HD))
  ones = jnp.ones((MBLK, 1), jnp.float32)
  for h in range(H):
    hf = slice(h * HF, (h + 1) * HF)
    q_ref[0, h, :, :HF] = qr1[:, hf] * scale
    q_ref[0, h, :, HF:] = qr2[:, hf] * scale
    k_ref[0, h, :, :HF] = kr1[:, hf]
    k_ref[0, h, :, HF:] = kr2[:, hf]
    v_ref[0, h, :, :HD] = v[:, h * HD:(h + 1) * HD]
    v_ref[0, h, :, HD:HD + 1] = ones


def _qkv_body(x_ref, wq_ref, wk_ref, wv_ref, bq_ref, bk_ref, bv_ref,
              sin_ref, cos_ref, q_ref, k_ref, v_ref):
  _qkv_compute(x_ref[...], wq_ref, wk_ref, wv_ref, bq_ref, bk_ref, bv_ref,
               sin_ref, cos_ref, q_ref, k_ref, v_ref)


def _qkv(x, wq_p, wk_p, wv, bq_p, bk_p, bv, sin_f, cos_f):
  hshape = jax.ShapeDtypeStruct((B, H, S, HD), jnp.float32)
  vshape = jax.ShapeDtypeStruct((B, H, S, 2 * HD), jnp.float32)
  mspec = lambda n: pl.BlockSpec((MBLK, n), lambda i: (i, 0))
  wspec = pl.BlockSpec((D, D), lambda i: (0, 0))
  bspec = pl.BlockSpec((1, D), lambda i: (0, 0))
  NQB = S // MBLK
  ospec = pl.BlockSpec((1, H, MBLK, HD), lambda i: (i // NQB, 0, i % NQB, 0))
  vspec = pl.BlockSpec((1, H, MBLK, 2 * HD),
                       lambda i: (i // NQB, 0, i % NQB, 0))
  return pl.pallas_call(
      _qkv_body,
      grid=(NMB,),
      in_specs=[mspec(D), wspec, wspec, wspec, bspec, bspec, bspec,
                mspec(D // 2), mspec(D // 2)],
      out_specs=[ospec, ospec, vspec],
      out_shape=[hshape, hshape, vshape],
      compiler_params=pltpu.CompilerParams(
          dimension_semantics=("arbitrary",)),
  )(x, wq_p, wk_p, wv, bq_p, bk_p, bv, sin_f, cos_f)


def _attn_body(q_ref, k_ref, v_ref, o_ref):
  dn = (((1,), (1,)), ((), ()))
  for h in range(_HPG):
    s = lax.dot_general(q_ref[0, h], k_ref[0, h], dn,
                        preferred_element_type=jnp.float32)
    p = jnp.exp(s)
    ol = jnp.dot(p, v_ref[0, h], preferred_element_type=jnp.float32)
    o_ref[0, h] = ol[:, :HD] / ol[:, HD:HD + 1]


def _attention(q, k, v):
  QBLK = 1024
  nq = S // QBLK
  ng = H // _HPG
  qspec = pl.BlockSpec((1, _HPG, QBLK, HD), lambda b, g, i: (b, g, i, 0))
  kspec = pl.BlockSpec((1, _HPG, S, HD), lambda b, g, i: (b, g, 0, 0))
  vspec = pl.BlockSpec((1, _HPG, S, 2 * HD), lambda b, g, i: (b, g, 0, 0))
  ospec = pl.BlockSpec((1, _HPG, QBLK, HD), lambda b, g, i: (b, g, i, 0))
  return pl.pallas_call(
      _attn_body,
      grid=(B, ng, nq),
      in_specs=[qspec, kspec, vspec],
      out_specs=ospec,
      out_shape=jax.ShapeDtypeStruct((B, H, S, HD), jnp.float32),
      compiler_params=pltpu.CompilerParams(
          dimension_semantics=("arbitrary", "arbitrary", "arbitrary")),
  )(q, k, v)


def _ln(t, g, b):
  m = jnp.mean(t, axis=-1, keepdims=True)
  d = t - m
  var = jnp.mean(d * d, axis=-1, keepdims=True)
  return d * lax.rsqrt(var + 1e-5) * g + b


def _proj_router_body(a_ref, wo_ref, bo_ref, res_ref, g_ref, b_ref, wr_ref,
                      x1_ref, logits_ref, eidx_ref, gatek_ref,
                      destd_ref, destc_ref, carry_ref):
  i = pl.program_id(0)

  @pl.when(i == 0)
  def _():
    carry_ref[...] = jnp.zeros((1, E), jnp.float32)

  a = jnp.concatenate([a_ref[0, h] for h in range(H)], axis=-1)
  t = jnp.dot(a, wo_ref[...], preferred_element_type=jnp.float32)
  t = t + bo_ref[...] + res_ref[...]
  x1 = _ln(t, g_ref[...], b_ref[...])
  x1_ref[...] = x1

  logits = jnp.dot(x1, wr_ref[...], preferred_element_type=jnp.float32)
  logits_ref[...] = logits
  m = jnp.max(logits, axis=-1, keepdims=True)
  p = jnp.exp(logits - m)
  p = p / jnp.sum(p, axis=-1, keepdims=True)
  gate = jnp.max(p, axis=-1, keepdims=True)
  iota_e = lax.broadcasted_iota(jnp.int32, (MBLK, E), 1)
  eidx = jnp.min(jnp.where(p == gate, iota_e, E), axis=-1,
                 keepdims=True)
  eidx_ref[...] = eidx
  oh = (iota_e == eidx).astype(jnp.float32)

  r = lax.broadcasted_iota(jnp.int32, (MBLK, MBLK), 0)
  cc = lax.broadcasted_iota(jnp.int32, (MBLK, MBLK), 1)
  tril = (r >= cc).astype(jnp.float32)
  cum = jnp.dot(tril, oh, preferred_element_type=jnp.float32) + carry_ref[...]
  carry_ref[...] = carry_ref[...] + jnp.sum(oh, axis=0, keepdims=True)

  pos = (jnp.sum(cum * oh, axis=-1, keepdims=True) - 1.0).astype(jnp.int32)
  keep = pos < CAP
  gatek_ref[...] = gate * keep.astype(jnp.float32)
  slot = eidx * CAP + pos
  tok = i * MBLK + lax.broadcasted_iota(jnp.int32, (MBLK, 1), 0)
  trash = T + tok // _SC_BPW
  destd_ref[...] = jnp.where(keep, slot, trash)
  destc_ref[...] = eidx * CAP + jnp.minimum(pos, CAP - 1)


def _proj_router(a, wo, bo, res, g, b, wr):
  NQB = S // MBLK
  aspec = pl.BlockSpec((1, H, MBLK, HD), lambda i: (i // NQB, 0, i % NQB, 0))
  mspec = pl.BlockSpec((MBLK, D), lambda i: (i, 0))
  espec = pl.BlockSpec((MBLK, E), lambda i: (i, 0))
  sspec = pl.BlockSpec((MBLK, 1), lambda i: (i, 0))
  wspec = pl.BlockSpec((D, D), lambda i: (0, 0))
  bspec = pl.BlockSpec((1, D), lambda i: (0, 0))
  rspec = pl.BlockSpec((D, E), lambda i: (0, 0))
  full = lambda shp, dt: jax.ShapeDtypeStruct(shp, dt)
  return pl.pallas_call(
      _proj_router_body,
      grid=(NMB,),
      in_specs=[aspec, wspec, bspec, mspec, bspec, bspec, rspec],
      out_specs=[mspec, espec, sspec, sspec, sspec, sspec],
      out_shape=[full((T, D), jnp.float32), full((T, E), jnp.float32),
                 full((T, 1), jnp.int32), full((T, 1), jnp.float32),
                 full((T, 1), jnp.int32), full((T, 1), jnp.int32)],
      scratch_shapes=[pltpu.VMEM((1, E), jnp.float32)],
      compiler_params=pltpu.CompilerParams(
          dimension_semantics=("arbitrary",)),
  )(a, wo, bo, res, g, b, wr)


def _gelu(x):
  return x * 0.5 * (1.0 + lax.erf(x * np.float32(1.0 / np.sqrt(2.0))))


def _ffn_body(xe_ref, w1_ref, b1_ref, w2_ref, b2_ref, ye_ref):
  f = pl.program_id(1)
  h = _gelu(jnp.dot(xe_ref[...].astype(w1_ref.dtype), w1_ref[0],
                    preferred_element_type=jnp.float32) + b1_ref[0])
  acc = jnp.dot(h.astype(w2_ref.dtype), w2_ref[0],
                preferred_element_type=jnp.float32)

  @pl.when(f == 0)
  def _():
    ye_ref[0] = acc + b2_ref[0]

  @pl.when(f != 0)
  def _():
    ye_ref[0] = ye_ref[0] + acc


def _ffn(xe, w1, b1, w2, b2):
  FBLK = 1536
  nf = F // FBLK
  return pl.pallas_call(
      _ffn_body,
      grid=(E, nf),
      in_specs=[
          pl.BlockSpec((CAP, D), lambda e, f: (e, 0)),
          pl.BlockSpec((1, D, FBLK), lambda e, f: (e, 0, f)),
          pl.BlockSpec((1, 1, FBLK), lambda e, f: (e, 0, f)),
          pl.BlockSpec((1, FBLK, D), lambda e, f: (e, f, 0)),
          pl.BlockSpec((1, 1, D), lambda e, f: (e, 0, 0)),
      ],
      out_specs=pl.BlockSpec((1, CAP, D), lambda e, f: (e, 0, 0)),
      out_shape=jax.ShapeDtypeStruct((E, CAP, D), jnp.float32),
      compiler_params=pltpu.CompilerParams(
          dimension_semantics=("arbitrary", "arbitrary")),
  )(xe, w1, b1, w2, b2)


def _combine_qkv_body(y_ref, gk_ref, res_ref, g_ref, b_ref,
                      wq_ref, wk_ref, wv_ref, bq_ref, bk_ref, bv_ref,
                      sin_ref, cos_ref, x_ref, q_ref, k_ref, v_ref):
  t = res_ref[...] + y_ref[...] * gk_ref[...]
  x = _ln(t, g_ref[...], b_ref[...])
  x_ref[...] = x
  _qkv_compute(x, wq_ref, wk_ref, wv_ref, bq_ref, bk_ref, bv_ref,
               sin_ref, cos_ref, q_ref, k_ref, v_ref)


def _combine_qkv(y, gk, res, g, b, wq_p, wk_p, wv, bq_p, bk_p, bv,
                 sin_f, cos_f):
  hshape = jax.ShapeDtypeStruct((B, H, S, HD), jnp.float32)
  vshape = jax.ShapeDtypeStruct((B, H, S, 2 * HD), jnp.float32)
  mspec = lambda n: pl.BlockSpec((MBLK, n), lambda i: (i, 0))
  gkspec = pl.BlockSpec((MBLK, 1), lambda i: (i, 0))
  wspec = pl.BlockSpec((D, D), lambda i: (0, 0))
  bspec = pl.BlockSpec((1, D), lambda i: (0, 0))
  NQB = S // MBLK
  ospec = pl.BlockSpec((1, H, MBLK, HD), lambda i: (i // NQB, 0, i % NQB, 0))
  vspec = pl.BlockSpec((1, H, MBLK, 2 * HD),
                       lambda i: (i // NQB, 0, i % NQB, 0))
  return pl.pallas_call(
      _combine_qkv_body,
      grid=(NMB,),
      in_specs=[mspec(D), gkspec, mspec(D), bspec, bspec,
                wspec, wspec, wspec, bspec, bspec, bspec,
                mspec(D // 2), mspec(D // 2)],
      out_specs=[mspec(D), ospec, ospec, vspec],
      out_shape=[jax.ShapeDtypeStruct((T, D), jnp.float32),
                 hshape, hshape, vshape],
      compiler_params=pltpu.CompilerParams(
          dimension_semantics=("arbitrary",)),
  )(y, gk, res, g, b, wq_p, wk_p, wv, bq_p, bk_p, bv, sin_f, cos_f)


def _combine_final_loss_body(y_ref, gk_ref, res_ref, g_ref, b_ref,
                             gf_ref, bf_ref, l1_ref, l2_ref, e1_ref, e2_ref,
                             o_ref, z_ref, aux_ref, acc_ref):
  i = pl.program_id(0)

  @pl.when(i == 0)
  def _():
    acc_ref[...] = jnp.zeros((2, E), jnp.float32)

  t = res_ref[...] + y_ref[...] * gk_ref[...]
  t = _ln(t, g_ref[...], b_ref[...])
  o_ref[...] = _ln(t, gf_ref[...], bf_ref[...])

  zsum = jnp.float32(0.0)
  psum = jnp.zeros((1, E), jnp.float32)
  msum = jnp.zeros((1, E), jnp.float32)
  for l_ref, e_ref in ((l1_ref, e1_ref), (l2_ref, e2_ref)):
    logits = l_ref[...]
    m = jnp.max(logits, axis=-1, keepdims=True)
    ex = jnp.exp(logits - m)
    se = jnp.sum(ex, axis=-1, keepdims=True)
    lse = m + jnp.log(se)
    zsum = zsum + jnp.sum(lse * lse)
    psum = psum + jnp.sum(ex / se, axis=0, keepdims=True)
    iota_e = lax.broadcasted_iota(jnp.int32, (MBLK, E), 1)
    msum = msum + jnp.sum((iota_e == e_ref[...]).astype(jnp.float32),
                          axis=0, keepdims=True)
  acc_ref[0:1, :] = acc_ref[0:1, :] + psum
  acc_ref[1:2, :] = acc_ref[1:2, :] + msum
  zprev = jnp.where(i == 0, 0.0, z_ref[0, 0])
  z_ref[...] = (zprev + zsum).reshape(1, 1)

  @pl.when(i == NMB - 1)
  def _():
    n = jnp.float32(L * T)
    z_ref[...] = (z_ref[0, 0] / n).reshape(1, 1)
    pe = acc_ref[0:1, :] / n
    me = acc_ref[1:2, :] / n
    aux_ref[...] = (jnp.sum(me * pe) * ((E * E) / E)).reshape(1, 1)


def _combine_final_loss(y, gk, res, g, b, gf, bf, l1, l2, e1, e2):
  mspec = pl.BlockSpec((MBLK, D), lambda i: (i, 0))
  gkspec = pl.BlockSpec((MBLK, 1), lambda i: (i, 0))
  espec = pl.BlockSpec((MBLK, E), lambda i: (i, 0))
  bspec = pl.BlockSpec((1, D), lambda i: (0, 0))
  sspec = pl.BlockSpec((1, 1), lambda i: (0, 0))
  return pl.pallas_call(
      _combine_final_loss_body,
      grid=(NMB,),
      in_specs=[mspec, gkspec, mspec, bspec, bspec, bspec, bspec,
                espec, espec, gkspec, gkspec],
      out_specs=[mspec, sspec, sspec],
      out_shape=[jax.ShapeDtypeStruct((T, D), jnp.float32),
                 jax.ShapeDtypeStruct((1, 1), jnp.float32),
                 jax.ShapeDtypeStruct((1, 1), jnp.float32)],
      scratch_shapes=[pltpu.VMEM((2, E), jnp.float32)],
      compiler_params=pltpu.CompilerParams(
          dimension_semantics=("arbitrary",)),
  )(y, gk, res, g, b, gf, bf, l1, l2, e1, e2)


def _rope_tables():
  inv = 1.0 / (10000.0 ** (np.arange(0, HD, 2, dtype=np.float64) / HD))
  ang = np.arange(S, dtype=np.float64)[:, None] * inv[None, :]
  sin = np.asarray(np.sin(ang), np.float32)
  cos = np.asarray(np.cos(ang), np.float32)
  sin_f = np.tile(np.tile(sin, (1, H)), (B, 1))
  cos_f = np.tile(np.tile(cos, (1, H)), (B, 1))
  return jnp.asarray(sin_f), jnp.asarray(cos_f)


_EVEN_ODD_PERM = np.concatenate([
    np.arange(D).reshape(H, HD)[:, 0::2].reshape(-1),
    np.arange(D).reshape(H, HD)[:, 1::2].reshape(-1),
])


def kernel(input_ids, params):
  p = params
  ids = input_ids.reshape(T).astype(jnp.int32)
  sin_f, cos_f = _rope_tables()

  x = _sc_gather(p['embed'], ids)

  def layer_w(l):
    wq_p = p['Wq'][l][:, _EVEN_ODD_PERM]
    wk_p = p['Wk'][l][:, _EVEN_ODD_PERM]
    bq_p = p['bq'][l][_EVEN_ODD_PERM].reshape(1, D)
    bk_p = p['bk'][l][_EVEN_ODD_PERM].reshape(1, D)
    return (wq_p, wk_p, p['Wv'][l], bq_p, bk_p, p['bv'][l].reshape(1, D),
            sin_f, cos_f)

  q, k, v = _qkv(x, *layer_w(0))
  a = _attention(q, k, v)
  x1, logits0, eidx0, gatek0, destd0, destc0 = _proj_router(
      a, p['Wo'][0], p['bo'][0].reshape(1, D), x,
      p['ln1_g'][0].reshape(1, D), p['ln1_b'][0].reshape(1, D), p['Wr'][0])
  xe = _sc_scatter(x1, destd0.reshape(T), T + _SC_NW)
  ye = _ffn(xe, p['W1'][0],
            p['b1'][0].reshape(E, 1, F), p['W2'][0],
            p['b2'][0].reshape(E, 1, D))
  y = _sc_gather(ye.reshape(E * CAP, D), destc0.reshape(T))

  x2, q, k, v = _combine_qkv(y, gatek0, x1, p['ln2_g'][0].reshape(1, D),
                             p['ln2_b'][0].reshape(1, D), *layer_w(1))
  a = _attention(q, k, v)
  x3, logits1, eidx1, gatek1, destd1, destc1 = _proj_router(
      a, p['Wo'][1], p['bo'][1].reshape(1, D), x2,
      p['ln1_g'][1].reshape(1, D), p['ln1_b'][1].reshape(1, D), p['Wr'][1])
  xe = _sc_scatter(x3, destd1.reshape(T), T + _SC_NW)
  ye = _ffn(xe, p['W1'][1],
            p['b1'][1].reshape(E, 1, F), p['W2'][1],
            p['b2'][1].reshape(E, 1, D))
  y = _sc_gather(ye.reshape(E * CAP, D), destc1.reshape(T))

  xout, z, aux = _combine_final_loss(
      y, gatek1, x3, p['ln2_g'][1].reshape(1, D), p['ln2_b'][1].reshape(1, D),
      p['final_g'].reshape(1, D), p['final_b'].reshape(1, D),
      logits0, logits1, eidx0, eidx1)
  return xout.reshape(B, S, D), z[0, 0], aux[0, 0]
